# baseline (speedup 1.0000x reference)
import functools
import math

import jax
import jax.numpy as jnp
from jax import lax
from jax.experimental import pallas as pl
from jax.experimental.pallas import tpu as pltpu
from jax.experimental.pallas import tpu_sc as plsc

N = 10000
E = 640000
D_IN = 128
H = 128
OUT = 768
VOCAB = 1000
G = 16
EPS = 1e-5

NC = 2
NS = 16
NW = NC * NS
CH = 128

E_PAD = 663552
EPW = E_PAD // NW
NCHUNK = EPW // CH

NID_PAD = 12288
ROWS_PER_W = NID_PAD // NW

ACC_ROWS = 10112
RPT = ACC_ROWS // NS
DUMMY = N

def _sc_mesh():
    return plsc.VectorSubcoreMesh(
        core_axis_name="c", subcore_axis_name="s",
        num_cores=NC, num_subcores=NS)


@functools.cache
def _get_sc_pre():
    return functools.partial(
        pl.kernel,
        out_type=(
            jax.ShapeDtypeStruct((NID_PAD, H), jnp.float32),
            jax.ShapeDtypeStruct((NC, ACC_ROWS, H), jnp.float32),
        ),
        mesh=_sc_mesh(),
        scratch_types=[
            pltpu.VMEM((CH,), jnp.int32),
            pltpu.VMEM((CH,), jnp.int32),
            pltpu.VMEM((CH,), jnp.int32),
            pltpu.VMEM((CH, H), jnp.float32),
            pltpu.VMEM_SHARED((ACC_ROWS, H), jnp.float32),
            pltpu.SemaphoreType.DMA,
            pltpu.SemaphoreType.DMA,
            pltpu.SemaphoreType.DMA,
        ],
    )(_sc_pre_body)


def _sc_pre_body(ids_hbm, dst_hbm, oh_hbm, z_hbm, emb_hbm,
                 embrows_hbm, deg_hbm,
                 di0, di1, ei_v, buf_v, acc_s, egsem, ds0, ds1):
    didx = [di0, di1]
    dsem = [ds0, ds1]
    ci = lax.axis_index("c")
    si = lax.axis_index("s")
    w = si * NC + ci
    row0 = si * RPT

    pltpu.sync_copy(z_hbm, buf_v)
    for k in range(4):
        pltpu.sync_copy(buf_v, acc_s.at[pl.ds(row0 + k * CH, CH)])
    pltpu.sync_copy(buf_v.at[pl.ds(0, RPT - 4 * CH)],
                    acc_s.at[pl.ds(row0 + 4 * CH, RPT - 4 * CH)])

    gbase = w * ROWS_PER_W
    for k in range(ROWS_PER_W // CH):
        b = gbase + k * CH
        pltpu.sync_copy(ids_hbm.at[pl.ds(b, CH)], ei_v)
        pltpu.async_copy(emb_hbm.at[ei_v], buf_v, egsem).wait()
        pltpu.sync_copy(buf_v, embrows_hbm.at[pl.ds(b, CH)])

    pltpu.sync_copy(oh_hbm, buf_v)
    plsc.subcore_barrier()

    ebase = w * EPW
    pltpu.sync_copy(dst_hbm.at[pl.ds(ebase, CH)], didx[0])

    def body(c0, carry):
        for bb in range(2):
            c = c0 * 2 + bb
            pltpu.async_copy(buf_v, acc_s.at[didx[bb]], dsem[bb], add=True)

            @pl.when(c + 1 < NCHUNK)
            def _():
                bq = (bb + 1) % 2

                @pl.when(c >= 1)
                def _():
                    pltpu.make_async_copy(buf_v, acc_s.at[didx[bq]],
                                          dsem[bq]).wait()

                pltpu.sync_copy(dst_hbm.at[pl.ds(ebase + (c + 1) * CH, CH)],
                                didx[bq])
        return carry

    lax.fori_loop(0, NCHUNK // 2, body, 0)
    for b in ((NCHUNK - 2) % 2, (NCHUNK - 1) % 2):
        pltpu.make_async_copy(buf_v, acc_s.at[didx[b]], dsem[b]).wait()

    plsc.subcore_barrier()
    for k in range(4):
        r = row0 + k * CH
        pltpu.sync_copy(acc_s.at[pl.ds(r, CH)], deg_hbm.at[ci, pl.ds(r, CH)])
    r = row0 + 4 * CH
    pltpu.sync_copy(acc_s.at[pl.ds(r, RPT - 4 * CH)],
                    deg_hbm.at[ci, pl.ds(r, RPT - 4 * CH)])


@functools.cache
def _get_sc_spmm():
    return functools.partial(
        pl.kernel,
        out_type=jax.ShapeDtypeStruct((NC, ACC_ROWS, H), jnp.float32),
        mesh=_sc_mesh(),
        scratch_types=(
            [pltpu.VMEM((CH,), jnp.int32)] * 3 +
            [pltpu.VMEM((CH,), jnp.int32)] * 3 +
            [pltpu.VMEM((CH, H), jnp.float32)] * 3 +
            [pltpu.VMEM_SHARED((ACC_ROWS, H), jnp.float32)] +
            [pltpu.SemaphoreType.DMA] * 6
        ),
    )(_sc_spmm_body)


def _sc_spmm_body(src_hbm, dst_hbm, z_hbm, table_hbm, parts_hbm,
                  si0, si1, si2, di0, di1, di2,
                  r0, r1, r2, acc_s,
                  g0, g1, g2, s0, s1, s2):
    sidx = [si0, si1, si2]
    didx = [di0, di1, di2]
    rows = [r0, r1, r2]
    gsem = [g0, g1, g2]
    ssem = [s0, s1, s2]
    ci = lax.axis_index("c")
    si = lax.axis_index("s")
    w = si * NC + ci
    row0 = si * RPT

    pltpu.sync_copy(z_hbm, rows[0])
    for k in range(4):
        pltpu.sync_copy(rows[0], acc_s.at[pl.ds(row0 + k * CH, CH)])
    pltpu.sync_copy(rows[0].at[pl.ds(0, RPT - 4 * CH)],
                    acc_s.at[pl.ds(row0 + 4 * CH, RPT - 4 * CH)])

    plsc.subcore_barrier()

    ebase = w * EPW

    def fetch(c, b):
        pltpu.sync_copy(src_hbm.at[pl.ds(ebase + c * CH, CH)], sidx[b])
        pltpu.sync_copy(dst_hbm.at[pl.ds(ebase + c * CH, CH)], didx[b])
        pltpu.async_copy(table_hbm.at[sidx[b]], rows[b], gsem[b])

    fetch(0, 0)
    fetch(1, 1)

    def body(c0, carry):
        for bb in range(3):
            c = c0 * 3 + bb
            pltpu.make_async_copy(table_hbm.at[sidx[bb]], rows[bb],
                                  gsem[bb]).wait()
            pltpu.async_copy(rows[bb], acc_s.at[didx[bb]], ssem[bb], add=True)

            @pl.when(c + 2 < NCHUNK)
            def _():
                bq = (bb + 2) % 3

                @pl.when(c >= 1)
                def _():
                    pltpu.make_async_copy(rows[bq], acc_s.at[didx[bq]],
                                          ssem[bq]).wait()

                fetch(c + 2, bq)
        return carry

    lax.fori_loop(0, NCHUNK // 3, body, 0)
    for b in ((NCHUNK - 3) % 3, (NCHUNK - 2) % 3, (NCHUNK - 1) % 3):
        pltpu.make_async_copy(rows[b], acc_s.at[didx[b]], ssem[b]).wait()

    plsc.subcore_barrier()
    for k in range(4):
        r = row0 + k * CH
        pltpu.sync_copy(acc_s.at[pl.ds(r, CH)], parts_hbm.at[ci, pl.ds(r, CH)])
    r = row0 + 4 * CH
    pltpu.sync_copy(acc_s.at[pl.ds(r, RPT - 4 * CH)],
                    parts_hbm.at[ci, pl.ds(r, RPT - 4 * CH)])


_BLK = 1000


def _dot(a, b):
    return lax.dot_general(a, b, (((1,), (0,)), ((), ())),
                           precision=lax.Precision.HIGHEST,
                           preferred_element_type=jnp.float32)


def _dis_of(d_r):
    return lax.rsqrt(1.0 + d_r[...])


def _tc_init_body(x_r, er_r, d_r, wp_r, bp_r, w0_r, h_r, hwp_r):
    dis = _dis_of(d_r)
    h = _dot(x_r[...], wp_r[...]) + bp_r[...] + er_r[...]
    h_r[...] = h
    hwp_r[...] = _dot(h, w0_r[...]) * dis


def _tc_layer_body(h_r, hwp_r, p0_r, p1_r, d_r, r_r, rb_r, sc_r, sh_r,
                   wn_r, h2_r, hwp2_r):
    dis = _dis_of(d_r)
    s = p0_r[...] + p1_r[...] + hwp_r[...]
    z = jnp.maximum(dis * s * sc_r[...] + sh_r[...], 0.0)
    h2 = _dot(h_r[...], r_r[...]) + rb_r[...] + z
    h2_r[...] = h2
    hwp2_r[...] = _dot(h2, wn_r[...]) * dis


def _tc_last_body(h_r, hwp_r, p0_r, p1_r, d_r, r_r, rb_r, sc_r, sh_r,
                  h2_r):
    dis = _dis_of(d_r)
    s = p0_r[...] + p1_r[...] + hwp_r[...]
    z = jnp.maximum(dis * s * sc_r[...] + sh_r[...], 0.0)
    h2_r[...] = _dot(h_r[...], r_r[...]) + rb_r[...] + z


def _tc_pool_body(h_r, b_r, wout_r, bout_r, lng_r, lnb_r, out_r,
                  sums, maxs, cnts):
    i = pl.program_id(0)

    @pl.when(i == 0)
    def _init():
        sums[...] = jnp.zeros_like(sums)
        cnts[...] = jnp.zeros_like(cnts)
        maxs[...] = jnp.full_like(maxs, -jnp.inf)

    b = b_r[...]
    h = h_r[...]
    oh = (b == lax.broadcasted_iota(jnp.int32, (1, G), 1)).astype(jnp.float32)
    contract = (((0,), (0,)), ((), ()))
    sums[...] += lax.dot_general(oh, h, contract,
                                 precision=lax.Precision.HIGHEST,
                                 preferred_element_type=jnp.float32)
    cnts[...] += lax.dot_general(oh, jnp.ones_like(h), contract,
                                 precision=lax.Precision.HIGHEST,
                                 preferred_element_type=jnp.float32)
    blockmax = jnp.concatenate(
        [jnp.max(jnp.where(b == g, h, -jnp.inf), axis=0, keepdims=True)
         for g in range(G)], axis=0)
    maxs[...] = jnp.maximum(maxs[...], blockmax)

    @pl.when(i == pl.num_programs(0) - 1)
    def _fin():
        mean = sums[...] / jnp.maximum(cnts[...], 1.0)
        ge = jnp.concatenate([mean, maxs[...]], axis=1)
        y = _dot(ge, wout_r[...]) + bout_r[...]
        y = jnp.maximum(y, 0.0)
        mu = jnp.mean(y, axis=1, keepdims=True)
        var = jnp.mean((y - mu) ** 2, axis=1, keepdims=True)
        out_r[...] = (y - mu) * lax.rsqrt(var + EPS) * lng_r[...] + lnb_r[...]


def _row_spec(cols):
    return pl.BlockSpec((_BLK, cols), lambda i: (i, 0))


def _full_spec(rows, cols):
    return pl.BlockSpec((rows, cols), lambda i: (0, 0))


def _tc_init(x, embrows, dcol, Wp, bp, W0):
    return pl.pallas_call(
        _tc_init_body,
        grid=(N // _BLK,),
        in_specs=[_row_spec(H), _row_spec(H), _row_spec(1),
                  _full_spec(D_IN, H), _full_spec(1, H), _full_spec(H, H)],
        out_specs=[_row_spec(H), _row_spec(H)],
        out_shape=[jax.ShapeDtypeStruct((N, H), jnp.float32)] * 2,
    )(x, embrows, dcol, Wp, bp, W0)


def _tc_layer(h, hwp, p0, p1, dcol, R, rb, scale, shift, Wn):
    return pl.pallas_call(
        _tc_layer_body,
        grid=(N // _BLK,),
        in_specs=[_row_spec(H)] * 4 + [_row_spec(1)] +
                 [_full_spec(H, H), _full_spec(1, H), _full_spec(1, H),
                  _full_spec(1, H), _full_spec(H, H)],
        out_specs=[_row_spec(H), _row_spec(H)],
        out_shape=[jax.ShapeDtypeStruct((N, H), jnp.float32)] * 2,
    )(h, hwp, p0, p1, dcol, R, rb, scale, shift, Wn)


def _tc_last(h, hwp, p0, p1, dcol, R, rb, scale, shift):
    return pl.pallas_call(
        _tc_last_body,
        grid=(N // _BLK,),
        in_specs=[_row_spec(H)] * 4 + [_row_spec(1)] +
                 [_full_spec(H, H), _full_spec(1, H), _full_spec(1, H),
                  _full_spec(1, H)],
        out_specs=[_row_spec(H)],
        out_shape=[jax.ShapeDtypeStruct((N, H), jnp.float32)],
    )(h, hwp, p0, p1, dcol, R, rb, scale, shift)[0]


def _tc_pool(h, batch2d, Wout, bout, ln_g, ln_b):
    return pl.pallas_call(
        _tc_pool_body,
        grid=(N // _BLK,),
        in_specs=[_row_spec(H), _row_spec(1),
                  _full_spec(2 * H, OUT), _full_spec(1, OUT),
                  _full_spec(1, OUT), _full_spec(1, OUT)],
        out_specs=[_full_spec(G, OUT)],
        out_shape=[jax.ShapeDtypeStruct((G, OUT), jnp.float32)],
        scratch_shapes=[pltpu.VMEM((G, H), jnp.float32),
                        pltpu.VMEM((G, H), jnp.float32),
                        pltpu.VMEM((G, H), jnp.float32)],
    )(h, batch2d, Wout, bout, ln_g, ln_b)[0]


def kernel(x, node_ids, edge_index, batch, emb, Wp, bp,
           W0, b0, G0, B0, R0, rb0,
           W1, b1, G1, B1, R1, rb1,
           W2, b2, G2, B2, R2, rb2,
           Wout, bout, ln_g, ln_b):
    f32 = jnp.float32
    src = edge_index[0].astype(jnp.int32)
    dst = edge_index[1].astype(jnp.int32)
    src_p = jnp.concatenate([src, jnp.zeros((E_PAD - E,), jnp.int32)])
    dst_p = jnp.concatenate([dst, jnp.full((E_PAD - E,), DUMMY, jnp.int32)])
    ids_p = jnp.concatenate(
        [node_ids.astype(jnp.int32), jnp.zeros((NID_PAD - N,), jnp.int32)])

    ohH = jnp.concatenate(
        [jnp.ones((CH, 1), f32), jnp.zeros((CH, H - 1), f32)], axis=1)
    zH = jnp.zeros((CH, H), f32)

    embrows, degp = _get_sc_pre()(ids_p, dst_p, ohH, zH, emb)
    embrows = embrows[:N]
    dcol = degp[0, :N, 0:1] + degp[1, :N, 0:1]

    cbn = 1.0 / math.sqrt(1.0 + EPS)
    bp2 = bp.reshape(1, H).astype(f32)
    scales = [(cbn * g).reshape(1, H) for g in (G0, G1, G2)]
    shifts = [(b * cbn * g + bb).reshape(1, H)
              for (b, g, bb) in ((b0, G0, B0), (b1, G1, B1), (b2, G2, B2))]

    h, hwp = _tc_init(x, embrows, dcol, Wp, bp2, W0)

    parts = _get_sc_spmm()(src_p, dst_p, zH, hwp)
    h, hwp = _tc_layer(h, hwp, parts[0, :N], parts[1, :N], dcol,
                       R0, rb0.reshape(1, H), scales[0], shifts[0], W1)

    parts = _get_sc_spmm()(src_p, dst_p, zH, hwp)
    h, hwp = _tc_layer(h, hwp, parts[0, :N], parts[1, :N], dcol,
                       R1, rb1.reshape(1, H), scales[1], shifts[1], W2)

    parts = _get_sc_spmm()(src_p, dst_p, zH, hwp)
    h = _tc_last(h, hwp, parts[0, :N], parts[1, :N], dcol,
                 R2, rb2.reshape(1, H), scales[2], shifts[2])

    batch2d = batch.astype(jnp.int32).reshape(N, 1)
    return _tc_pool(h, batch2d, Wout, bout.reshape(1, OUT),
                    ln_g.reshape(1, OUT), ln_b.reshape(1, OUT))


# speedup vs baseline: 1.0080x; 1.0080x over previous
import functools
import math

import jax
import jax.numpy as jnp
from jax import lax
from jax.experimental import pallas as pl
from jax.experimental.pallas import tpu as pltpu
from jax.experimental.pallas import tpu_sc as plsc

N = 10000
E = 640000
D_IN = 128
H = 128
OUT = 768
VOCAB = 1000
G = 16
EPS = 1e-5

NC = 2
NS = 16
NW = NC * NS
CH = 128

E_PAD = 663552
EPW = E_PAD // NW
NCHUNK = EPW // CH

NID_PAD = 12288
ROWS_PER_W = NID_PAD // NW

ACC_ROWS = 10112
RPT = ACC_ROWS // NS
DUMMY = N

def _sc_mesh():
    return plsc.VectorSubcoreMesh(
        core_axis_name="c", subcore_axis_name="s",
        num_cores=NC, num_subcores=NS)


@functools.cache
def _get_sc_pre():
    return functools.partial(
        pl.kernel,
        out_type=(
            jax.ShapeDtypeStruct((NID_PAD, H), jnp.float32),
            jax.ShapeDtypeStruct((NC, ACC_ROWS, H), jnp.float32),
        ),
        mesh=_sc_mesh(),
        scratch_types=[
            pltpu.VMEM((CH,), jnp.int32),
            pltpu.VMEM((CH,), jnp.int32),
            pltpu.VMEM((CH,), jnp.int32),
            pltpu.VMEM((CH, H), jnp.float32),
            pltpu.VMEM_SHARED((ACC_ROWS, H), jnp.float32),
            pltpu.SemaphoreType.DMA,
            pltpu.SemaphoreType.DMA,
            pltpu.SemaphoreType.DMA,
        ],
    )(_sc_pre_body)


def _sc_pre_body(ids_hbm, dst_hbm, oh_hbm, z_hbm, emb_hbm,
                 embrows_hbm, deg_hbm,
                 di0, di1, ei_v, buf_v, acc_s, egsem, ds0, ds1):
    didx = [di0, di1]
    dsem = [ds0, ds1]
    ci = lax.axis_index("c")
    si = lax.axis_index("s")
    w = si * NC + ci
    row0 = si * RPT

    pltpu.sync_copy(z_hbm, buf_v)
    for k in range(4):
        pltpu.sync_copy(buf_v, acc_s.at[pl.ds(row0 + k * CH, CH)])
    pltpu.sync_copy(buf_v.at[pl.ds(0, RPT - 4 * CH)],
                    acc_s.at[pl.ds(row0 + 4 * CH, RPT - 4 * CH)])

    gbase = w * ROWS_PER_W
    for k in range(ROWS_PER_W // CH):
        b = gbase + k * CH
        pltpu.sync_copy(ids_hbm.at[pl.ds(b, CH)], ei_v)
        pltpu.async_copy(emb_hbm.at[ei_v], buf_v, egsem).wait()
        pltpu.sync_copy(buf_v, embrows_hbm.at[pl.ds(b, CH)])

    pltpu.sync_copy(oh_hbm, buf_v)
    plsc.subcore_barrier()

    ebase = w * EPW
    pltpu.sync_copy(dst_hbm.at[pl.ds(ebase, CH)], didx[0])

    def body(c0, carry):
        for bb in range(2):
            c = c0 * 2 + bb
            pltpu.async_copy(buf_v, acc_s.at[didx[bb]], dsem[bb], add=True)

            @pl.when(c + 1 < NCHUNK)
            def _():
                bq = (bb + 1) % 2

                @pl.when(c >= 1)
                def _():
                    pltpu.make_async_copy(buf_v, acc_s.at[didx[bq]],
                                          dsem[bq]).wait()

                pltpu.sync_copy(dst_hbm.at[pl.ds(ebase + (c + 1) * CH, CH)],
                                didx[bq])
        return carry

    lax.fori_loop(0, NCHUNK // 2, body, 0)
    for b in ((NCHUNK - 2) % 2, (NCHUNK - 1) % 2):
        pltpu.make_async_copy(buf_v, acc_s.at[didx[b]], dsem[b]).wait()

    plsc.subcore_barrier()
    for k in range(4):
        r = row0 + k * CH
        pltpu.sync_copy(acc_s.at[pl.ds(r, CH)], deg_hbm.at[ci, pl.ds(r, CH)])
    r = row0 + 4 * CH
    pltpu.sync_copy(acc_s.at[pl.ds(r, RPT - 4 * CH)],
                    deg_hbm.at[ci, pl.ds(r, RPT - 4 * CH)])


@functools.cache
def _get_sc_spmm():
    return functools.partial(
        pl.kernel,
        out_type=jax.ShapeDtypeStruct((NC, ACC_ROWS, H), jnp.float32),
        mesh=_sc_mesh(),
        scratch_types=(
            [pltpu.VMEM((CH,), jnp.int32)] * 3 +
            [pltpu.VMEM((CH,), jnp.int32)] * 3 +
            [pltpu.VMEM((CH, H), jnp.float32)] * 3 +
            [pltpu.VMEM_SHARED((ACC_ROWS, H), jnp.float32)] +
            [pltpu.SemaphoreType.DMA] * 3
        ),
    )(_sc_spmm_body)


def _sc_spmm_body(src_hbm, dst_hbm, z_hbm, table_hbm, parts_hbm,
                  si0, si1, si2, di0, di1, di2,
                  r0, r1, r2, acc_s, g0, g1, g2):
    sidx = [si0, si1, si2]
    didx = [di0, di1, di2]
    rows = [r0, r1, r2]
    gsem = [g0, g1, g2]
    ci = lax.axis_index("c")
    si = lax.axis_index("s")
    w = si * NC + ci
    row0 = si * RPT

    pltpu.sync_copy(z_hbm, rows[0])
    for k in range(4):
        pltpu.sync_copy(rows[0], acc_s.at[pl.ds(row0 + k * CH, CH)])
    pltpu.sync_copy(rows[0].at[pl.ds(0, RPT - 4 * CH)],
                    acc_s.at[pl.ds(row0 + 4 * CH, RPT - 4 * CH)])

    plsc.subcore_barrier()

    ebase = w * EPW

    def fetch(c, b):
        pltpu.sync_copy(src_hbm.at[pl.ds(ebase + c * CH, CH)], sidx[b])
        pltpu.sync_copy(dst_hbm.at[pl.ds(ebase + c * CH, CH)], didx[b])
        pltpu.async_copy(table_hbm.at[sidx[b]], rows[b], gsem[b])

    fetch(0, 0)
    fetch(1, 1)

    def body(c0, carry):
        for bb in range(3):
            c = c0 * 3 + bb

            @pl.when(c + 2 < NCHUNK)
            def _():
                fetch(c + 2, (bb + 2) % 3)

            pltpu.make_async_copy(table_hbm.at[sidx[bb]], rows[bb],
                                  gsem[bb]).wait()
            pltpu.sync_copy(rows[bb], acc_s.at[didx[bb]], add=True)
        return carry

    lax.fori_loop(0, NCHUNK // 3, body, 0)

    plsc.subcore_barrier()
    for k in range(4):
        r = row0 + k * CH
        pltpu.sync_copy(acc_s.at[pl.ds(r, CH)], parts_hbm.at[ci, pl.ds(r, CH)])
    r = row0 + 4 * CH
    pltpu.sync_copy(acc_s.at[pl.ds(r, RPT - 4 * CH)],
                    parts_hbm.at[ci, pl.ds(r, RPT - 4 * CH)])


_BLK = 1000


def _dot(a, b):
    return lax.dot_general(a, b, (((1,), (0,)), ((), ())),
                           precision=lax.Precision.HIGHEST,
                           preferred_element_type=jnp.float32)


def _dis_of(d_r):
    return lax.rsqrt(1.0 + d_r[...])


def _tc_init_body(x_r, er_r, d_r, wp_r, bp_r, w0_r, h_r, hwp_r):
    dis = _dis_of(d_r)
    h = _dot(x_r[...], wp_r[...]) + bp_r[...] + er_r[...]
    h_r[...] = h
    hwp_r[...] = _dot(h, w0_r[...]) * dis


def _tc_layer_body(h_r, hwp_r, p0_r, p1_r, d_r, r_r, rb_r, sc_r, sh_r,
                   wn_r, h2_r, hwp2_r):
    dis = _dis_of(d_r)
    s = p0_r[...] + p1_r[...] + hwp_r[...]
    z = jnp.maximum(dis * s * sc_r[...] + sh_r[...], 0.0)
    h2 = _dot(h_r[...], r_r[...]) + rb_r[...] + z
    h2_r[...] = h2
    hwp2_r[...] = _dot(h2, wn_r[...]) * dis


def _tc_last_body(h_r, hwp_r, p0_r, p1_r, d_r, r_r, rb_r, sc_r, sh_r,
                  h2_r):
    dis = _dis_of(d_r)
    s = p0_r[...] + p1_r[...] + hwp_r[...]
    z = jnp.maximum(dis * s * sc_r[...] + sh_r[...], 0.0)
    h2_r[...] = _dot(h_r[...], r_r[...]) + rb_r[...] + z


def _tc_pool_body(h_r, b_r, wout_r, bout_r, lng_r, lnb_r, out_r,
                  sums, maxs, cnts):
    i = pl.program_id(0)

    @pl.when(i == 0)
    def _init():
        sums[...] = jnp.zeros_like(sums)
        cnts[...] = jnp.zeros_like(cnts)
        maxs[...] = jnp.full_like(maxs, -jnp.inf)

    b = b_r[...]
    h = h_r[...]
    oh = (b == lax.broadcasted_iota(jnp.int32, (1, G), 1)).astype(jnp.float32)
    contract = (((0,), (0,)), ((), ()))
    sums[...] += lax.dot_general(oh, h, contract,
                                 precision=lax.Precision.HIGHEST,
                                 preferred_element_type=jnp.float32)
    cnts[...] += lax.dot_general(oh, jnp.ones_like(h), contract,
                                 precision=lax.Precision.HIGHEST,
                                 preferred_element_type=jnp.float32)
    blockmax = jnp.concatenate(
        [jnp.max(jnp.where(b == g, h, -jnp.inf), axis=0, keepdims=True)
         for g in range(G)], axis=0)
    maxs[...] = jnp.maximum(maxs[...], blockmax)

    @pl.when(i == pl.num_programs(0) - 1)
    def _fin():
        mean = sums[...] / jnp.maximum(cnts[...], 1.0)
        ge = jnp.concatenate([mean, maxs[...]], axis=1)
        y = _dot(ge, wout_r[...]) + bout_r[...]
        y = jnp.maximum(y, 0.0)
        mu = jnp.mean(y, axis=1, keepdims=True)
        var = jnp.mean((y - mu) ** 2, axis=1, keepdims=True)
        out_r[...] = (y - mu) * lax.rsqrt(var + EPS) * lng_r[...] + lnb_r[...]


def _row_spec(cols):
    return pl.BlockSpec((_BLK, cols), lambda i: (i, 0))


def _full_spec(rows, cols):
    return pl.BlockSpec((rows, cols), lambda i: (0, 0))


def _tc_init(x, embrows, dcol, Wp, bp, W0):
    return pl.pallas_call(
        _tc_init_body,
        grid=(N // _BLK,),
        in_specs=[_row_spec(H), _row_spec(H), _row_spec(1),
                  _full_spec(D_IN, H), _full_spec(1, H), _full_spec(H, H)],
        out_specs=[_row_spec(H), _row_spec(H)],
        out_shape=[jax.ShapeDtypeStruct((N, H), jnp.float32)] * 2,
    )(x, embrows, dcol, Wp, bp, W0)


def _tc_layer(h, hwp, p0, p1, dcol, R, rb, scale, shift, Wn):
    return pl.pallas_call(
        _tc_layer_body,
        grid=(N // _BLK,),
        in_specs=[_row_spec(H)] * 4 + [_row_spec(1)] +
                 [_full_spec(H, H), _full_spec(1, H), _full_spec(1, H),
                  _full_spec(1, H), _full_spec(H, H)],
        out_specs=[_row_spec(H), _row_spec(H)],
        out_shape=[jax.ShapeDtypeStruct((N, H), jnp.float32)] * 2,
    )(h, hwp, p0, p1, dcol, R, rb, scale, shift, Wn)


def _tc_last(h, hwp, p0, p1, dcol, R, rb, scale, shift):
    return pl.pallas_call(
        _tc_last_body,
        grid=(N // _BLK,),
        in_specs=[_row_spec(H)] * 4 + [_row_spec(1)] +
                 [_full_spec(H, H), _full_spec(1, H), _full_spec(1, H),
                  _full_spec(1, H)],
        out_specs=[_row_spec(H)],
        out_shape=[jax.ShapeDtypeStruct((N, H), jnp.float32)],
    )(h, hwp, p0, p1, dcol, R, rb, scale, shift)[0]


def _tc_pool(h, batch2d, Wout, bout, ln_g, ln_b):
    return pl.pallas_call(
        _tc_pool_body,
        grid=(N // _BLK,),
        in_specs=[_row_spec(H), _row_spec(1),
                  _full_spec(2 * H, OUT), _full_spec(1, OUT),
                  _full_spec(1, OUT), _full_spec(1, OUT)],
        out_specs=[_full_spec(G, OUT)],
        out_shape=[jax.ShapeDtypeStruct((G, OUT), jnp.float32)],
        scratch_shapes=[pltpu.VMEM((G, H), jnp.float32),
                        pltpu.VMEM((G, H), jnp.float32),
                        pltpu.VMEM((G, H), jnp.float32)],
    )(h, batch2d, Wout, bout, ln_g, ln_b)[0]


def kernel(x, node_ids, edge_index, batch, emb, Wp, bp,
           W0, b0, G0, B0, R0, rb0,
           W1, b1, G1, B1, R1, rb1,
           W2, b2, G2, B2, R2, rb2,
           Wout, bout, ln_g, ln_b):
    f32 = jnp.float32
    src = edge_index[0].astype(jnp.int32)
    dst = edge_index[1].astype(jnp.int32)
    src_p = jnp.concatenate([src, jnp.zeros((E_PAD - E,), jnp.int32)])
    dst_p = jnp.concatenate([dst, jnp.full((E_PAD - E,), DUMMY, jnp.int32)])
    ids_p = jnp.concatenate(
        [node_ids.astype(jnp.int32), jnp.zeros((NID_PAD - N,), jnp.int32)])

    ohH = jnp.concatenate(
        [jnp.ones((CH, 1), f32), jnp.zeros((CH, H - 1), f32)], axis=1)
    zH = jnp.zeros((CH, H), f32)

    embrows, degp = _get_sc_pre()(ids_p, dst_p, ohH, zH, emb)
    embrows = embrows[:N]
    dcol = degp[0, :N, 0:1] + degp[1, :N, 0:1]

    cbn = 1.0 / math.sqrt(1.0 + EPS)
    bp2 = bp.reshape(1, H).astype(f32)
    scales = [(cbn * g).reshape(1, H) for g in (G0, G1, G2)]
    shifts = [(b * cbn * g + bb).reshape(1, H)
              for (b, g, bb) in ((b0, G0, B0), (b1, G1, B1), (b2, G2, B2))]

    h, hwp = _tc_init(x, embrows, dcol, Wp, bp2, W0)

    parts = _get_sc_spmm()(src_p, dst_p, zH, hwp)
    h, hwp = _tc_layer(h, hwp, parts[0, :N], parts[1, :N], dcol,
                       R0, rb0.reshape(1, H), scales[0], shifts[0], W1)

    parts = _get_sc_spmm()(src_p, dst_p, zH, hwp)
    h, hwp = _tc_layer(h, hwp, parts[0, :N], parts[1, :N], dcol,
                       R1, rb1.reshape(1, H), scales[1], shifts[1], W2)

    parts = _get_sc_spmm()(src_p, dst_p, zH, hwp)
    h = _tc_last(h, hwp, parts[0, :N], parts[1, :N], dcol,
                 R2, rb2.reshape(1, H), scales[2], shifts[2])

    batch2d = batch.astype(jnp.int32).reshape(N, 1)
    return _tc_pool(h, batch2d, Wout, bout.reshape(1, OUT),
                    ln_g.reshape(1, OUT), ln_b.reshape(1, OUT))


# speedup vs baseline: 3.4109x; 3.3839x over previous
import functools
import math

import jax
import jax.numpy as jnp
from jax import lax
from jax.experimental import pallas as pl
from jax.experimental.pallas import tpu as pltpu
from jax.experimental.pallas import tpu_sc as plsc

N = 10000
E = 640000
D_IN = 128
H = 128
OUT = 768
VOCAB = 1000
G = 16
EPS = 1e-5

NC = 2
NS = 16
NW = NC * NS
CH = 128

E_PAD = 663552
EPW = E_PAD // NW
NCHUNK = EPW // CH

NID_PAD = 12288
ROWS_PER_W = NID_PAD // NW

ACC_ROWS = 10112
RPT = ACC_ROWS // NS
DUMMY = N

def _sc_mesh():
    return plsc.VectorSubcoreMesh(
        core_axis_name="c", subcore_axis_name="s",
        num_cores=NC, num_subcores=NS)


@functools.cache
def _get_sc_pre():
    return functools.partial(
        pl.kernel,
        out_type=(
            jax.ShapeDtypeStruct((NID_PAD, H), jnp.float32),
            jax.ShapeDtypeStruct((NC, ACC_ROWS, H), jnp.float32),
        ),
        mesh=_sc_mesh(),
        scratch_types=[
            pltpu.VMEM((CH,), jnp.int32),
            pltpu.VMEM((CH,), jnp.int32),
            pltpu.VMEM((CH,), jnp.int32),
            pltpu.VMEM((CH, H), jnp.float32),
            pltpu.VMEM_SHARED((ACC_ROWS, H), jnp.float32),
            pltpu.SemaphoreType.DMA,
            pltpu.SemaphoreType.DMA,
            pltpu.SemaphoreType.DMA,
        ],
    )(_sc_pre_body)


def _sc_pre_body(ids_hbm, dst_hbm, oh_hbm, z_hbm, emb_hbm,
                 embrows_hbm, deg_hbm,
                 di0, di1, ei_v, buf_v, acc_s, egsem, ds0, ds1):
    didx = [di0, di1]
    dsem = [ds0, ds1]
    ci = lax.axis_index("c")
    si = lax.axis_index("s")
    w = si * NC + ci
    row0 = si * RPT

    pltpu.sync_copy(z_hbm, buf_v)
    for k in range(4):
        pltpu.sync_copy(buf_v, acc_s.at[pl.ds(row0 + k * CH, CH)])
    pltpu.sync_copy(buf_v.at[pl.ds(0, RPT - 4 * CH)],
                    acc_s.at[pl.ds(row0 + 4 * CH, RPT - 4 * CH)])

    gbase = w * ROWS_PER_W
    for k in range(ROWS_PER_W // CH):
        b = gbase + k * CH
        pltpu.sync_copy(ids_hbm.at[pl.ds(b, CH)], ei_v)
        pltpu.async_copy(emb_hbm.at[ei_v], buf_v, egsem).wait()
        pltpu.sync_copy(buf_v, embrows_hbm.at[pl.ds(b, CH)])

    pltpu.sync_copy(oh_hbm, buf_v)
    plsc.subcore_barrier()

    ebase = w * EPW
    pltpu.sync_copy(dst_hbm.at[pl.ds(ebase, CH)], didx[0])

    def body(c0, carry):
        for bb in range(2):
            c = c0 * 2 + bb
            pltpu.async_copy(buf_v, acc_s.at[didx[bb]], dsem[bb], add=True)

            @pl.when(c + 1 < NCHUNK)
            def _():
                bq = (bb + 1) % 2

                @pl.when(c >= 1)
                def _():
                    pltpu.make_async_copy(buf_v, acc_s.at[didx[bq]],
                                          dsem[bq]).wait()

                pltpu.sync_copy(dst_hbm.at[pl.ds(ebase + (c + 1) * CH, CH)],
                                didx[bq])
        return carry

    lax.fori_loop(0, NCHUNK // 2, body, 0)
    for b in ((NCHUNK - 2) % 2, (NCHUNK - 1) % 2):
        pltpu.make_async_copy(buf_v, acc_s.at[didx[b]], dsem[b]).wait()

    plsc.subcore_barrier()
    for k in range(4):
        r = row0 + k * CH
        pltpu.sync_copy(acc_s.at[pl.ds(r, CH)], deg_hbm.at[ci, pl.ds(r, CH)])
    r = row0 + 4 * CH
    pltpu.sync_copy(acc_s.at[pl.ds(r, RPT - 4 * CH)],
                    deg_hbm.at[ci, pl.ds(r, RPT - 4 * CH)])


@functools.cache
def _get_sc_spmm():
    return functools.partial(
        pl.kernel,
        out_type=jax.ShapeDtypeStruct((NC, ACC_ROWS, H), jnp.float32),
        mesh=_sc_mesh(),
        scratch_types=(
            [pltpu.VMEM((CH,), jnp.int32)] * 3 +
            [pltpu.VMEM((CH,), jnp.int32)] * 3 +
            [pltpu.VMEM((CH, H), jnp.float32)] * 3 +
            [pltpu.VMEM_SHARED((ACC_ROWS, H), jnp.float32)] +
            [pltpu.SemaphoreType.DMA] * 3
        ),
    )(_sc_spmm_body)


def _sc_spmm_body(src_hbm, dst_hbm, z_hbm, table_hbm, parts_hbm,
                  si0, si1, si2, di0, di1, di2,
                  r0, r1, r2, acc_s, g0, g1, g2):
    sidx = [si0, si1, si2]
    didx = [di0, di1, di2]
    rows = [r0, r1, r2]
    gsem = [g0, g1, g2]
    ci = lax.axis_index("c")
    si = lax.axis_index("s")
    w = si * NC + ci
    row0 = si * RPT

    pltpu.sync_copy(z_hbm, rows[0])
    for k in range(4):
        pltpu.sync_copy(rows[0], acc_s.at[pl.ds(row0 + k * CH, CH)])
    pltpu.sync_copy(rows[0].at[pl.ds(0, RPT - 4 * CH)],
                    acc_s.at[pl.ds(row0 + 4 * CH, RPT - 4 * CH)])

    plsc.subcore_barrier()

    ebase = w * EPW

    def fetch(c, b):
        pltpu.sync_copy(src_hbm.at[pl.ds(ebase + c * CH, CH)], sidx[b])
        pltpu.sync_copy(dst_hbm.at[pl.ds(ebase + c * CH, CH)], didx[b])
        pltpu.async_copy(table_hbm.at[sidx[b]], rows[b], gsem[b])

    fetch(0, 0)
    fetch(1, 1)

    def body(c0, carry):
        for bb in range(3):
            c = c0 * 3 + bb

            @pl.when(c + 2 < NCHUNK)
            def _():
                fetch(c + 2, (bb + 2) % 3)

            pltpu.make_async_copy(table_hbm.at[sidx[bb]], rows[bb],
                                  gsem[bb]).wait()
            pltpu.sync_copy(rows[bb], acc_s.at[didx[bb]], add=True)
        return carry

    lax.fori_loop(0, NCHUNK // 3, body, 0)

    plsc.subcore_barrier()
    for k in range(4):
        r = row0 + k * CH
        pltpu.sync_copy(acc_s.at[pl.ds(r, CH)], parts_hbm.at[ci, pl.ds(r, CH)])
    r = row0 + 4 * CH
    pltpu.sync_copy(acc_s.at[pl.ds(r, RPT - 4 * CH)],
                    parts_hbm.at[ci, pl.ds(r, RPT - 4 * CH)])


_BLK = 1000


def _dot(a, b):
    return lax.dot_general(a, b, (((1,), (0,)), ((), ())),
                           precision=lax.Precision.HIGHEST,
                           preferred_element_type=jnp.float32)


def _dis_of(d_r):
    return lax.rsqrt(1.0 + d_r[...])


def _tc_init_body(x_r, er_r, d_r, wp_r, bp_r, w0_r, h_r, hwp_r):
    dis = _dis_of(d_r)
    h = _dot(x_r[...], wp_r[...]) + bp_r[...] + er_r[...]
    h_r[...] = h
    hwp_r[...] = _dot(h, w0_r[...]) * dis


def _tc_layer_body(h_r, hwp_r, p0_r, p1_r, d_r, r_r, rb_r, sc_r, sh_r,
                   wn_r, h2_r, hwp2_r):
    dis = _dis_of(d_r)
    s = p0_r[...] + p1_r[...] + hwp_r[...]
    z = jnp.maximum(dis * s * sc_r[...] + sh_r[...], 0.0)
    h2 = _dot(h_r[...], r_r[...]) + rb_r[...] + z
    h2_r[...] = h2
    hwp2_r[...] = _dot(h2, wn_r[...]) * dis


def _tc_last_body(h_r, hwp_r, p0_r, p1_r, d_r, r_r, rb_r, sc_r, sh_r,
                  h2_r):
    dis = _dis_of(d_r)
    s = p0_r[...] + p1_r[...] + hwp_r[...]
    z = jnp.maximum(dis * s * sc_r[...] + sh_r[...], 0.0)
    h2_r[...] = _dot(h_r[...], r_r[...]) + rb_r[...] + z


def _tc_pool_body(h_r, b_r, wout_r, bout_r, lng_r, lnb_r, out_r,
                  sums, maxs, cnts):
    i = pl.program_id(0)

    @pl.when(i == 0)
    def _init():
        sums[...] = jnp.zeros_like(sums)
        cnts[...] = jnp.zeros_like(cnts)
        maxs[...] = jnp.full_like(maxs, -jnp.inf)

    b = b_r[...]
    h = h_r[...]
    oh = (b == lax.broadcasted_iota(jnp.int32, (1, G), 1)).astype(jnp.float32)
    contract = (((0,), (0,)), ((), ()))
    sums[...] += lax.dot_general(oh, h, contract,
                                 precision=lax.Precision.HIGHEST,
                                 preferred_element_type=jnp.float32)
    cnts[...] += lax.dot_general(oh, jnp.ones_like(h), contract,
                                 precision=lax.Precision.HIGHEST,
                                 preferred_element_type=jnp.float32)
    blockmax = jnp.concatenate(
        [jnp.max(jnp.where(b == g, h, -jnp.inf), axis=0, keepdims=True)
         for g in range(G)], axis=0)
    maxs[...] = jnp.maximum(maxs[...], blockmax)

    @pl.when(i == pl.num_programs(0) - 1)
    def _fin():
        mean = sums[...] / jnp.maximum(cnts[...], 1.0)
        ge = jnp.concatenate([mean, maxs[...]], axis=1)
        y = _dot(ge, wout_r[...]) + bout_r[...]
        y = jnp.maximum(y, 0.0)
        mu = jnp.mean(y, axis=1, keepdims=True)
        var = jnp.mean((y - mu) ** 2, axis=1, keepdims=True)
        out_r[...] = (y - mu) * lax.rsqrt(var + EPS) * lng_r[...] + lnb_r[...]


def _row_spec(cols):
    return pl.BlockSpec((_BLK, cols), lambda i: (i, 0))


def _full_spec(rows, cols):
    return pl.BlockSpec((rows, cols), lambda i: (0, 0))


def _tc_init(x, embrows, dcol, Wp, bp, W0):
    return pl.pallas_call(
        _tc_init_body,
        grid=(N // _BLK,),
        in_specs=[_row_spec(H), _row_spec(H), _row_spec(1),
                  _full_spec(D_IN, H), _full_spec(1, H), _full_spec(H, H)],
        out_specs=[_row_spec(H), _row_spec(H)],
        out_shape=[jax.ShapeDtypeStruct((N, H), jnp.float32)] * 2,
    )(x, embrows, dcol, Wp, bp, W0)


def _tc_layer(h, hwp, p0, p1, dcol, R, rb, scale, shift, Wn):
    return pl.pallas_call(
        _tc_layer_body,
        grid=(N // _BLK,),
        in_specs=[_row_spec(H)] * 4 + [_row_spec(1)] +
                 [_full_spec(H, H), _full_spec(1, H), _full_spec(1, H),
                  _full_spec(1, H), _full_spec(H, H)],
        out_specs=[_row_spec(H), _row_spec(H)],
        out_shape=[jax.ShapeDtypeStruct((N, H), jnp.float32)] * 2,
    )(h, hwp, p0, p1, dcol, R, rb, scale, shift, Wn)


def _tc_last(h, hwp, p0, p1, dcol, R, rb, scale, shift):
    return pl.pallas_call(
        _tc_last_body,
        grid=(N // _BLK,),
        in_specs=[_row_spec(H)] * 4 + [_row_spec(1)] +
                 [_full_spec(H, H), _full_spec(1, H), _full_spec(1, H),
                  _full_spec(1, H)],
        out_specs=[_row_spec(H)],
        out_shape=[jax.ShapeDtypeStruct((N, H), jnp.float32)],
    )(h, hwp, p0, p1, dcol, R, rb, scale, shift)[0]


def _tc_pool(h, batch2d, Wout, bout, ln_g, ln_b):
    return pl.pallas_call(
        _tc_pool_body,
        grid=(N // _BLK,),
        in_specs=[_row_spec(H), _row_spec(1),
                  _full_spec(2 * H, OUT), _full_spec(1, OUT),
                  _full_spec(1, OUT), _full_spec(1, OUT)],
        out_specs=[_full_spec(G, OUT)],
        out_shape=[jax.ShapeDtypeStruct((G, OUT), jnp.float32)],
        scratch_shapes=[pltpu.VMEM((G, H), jnp.float32),
                        pltpu.VMEM((G, H), jnp.float32),
                        pltpu.VMEM((G, H), jnp.float32)],
    )(h, batch2d, Wout, bout, ln_g, ln_b)[0]


def kernel(x, node_ids, edge_index, batch, emb, Wp, bp,
           W0, b0, G0, B0, R0, rb0,
           W1, b1, G1, B1, R1, rb1,
           W2, b2, G2, B2, R2, rb2,
           Wout, bout, ln_g, ln_b):
    f32 = jnp.float32
    src = edge_index[0].astype(jnp.int32)
    dst = edge_index[1].astype(jnp.int32)
    pad_i = jnp.arange(E_PAD - E, dtype=jnp.int32)
    src_p = jnp.concatenate([src, pad_i % N])
    dst_p = jnp.concatenate([dst, DUMMY + pad_i % (ACC_ROWS - N)])
    ids_p = jnp.concatenate(
        [node_ids.astype(jnp.int32), jnp.zeros((NID_PAD - N,), jnp.int32)])

    ohH = jnp.concatenate(
        [jnp.ones((CH, 1), f32), jnp.zeros((CH, H - 1), f32)], axis=1)
    zH = jnp.zeros((CH, H), f32)

    embrows, degp = _get_sc_pre()(ids_p, dst_p, ohH, zH, emb)
    embrows = embrows[:N]
    dcol = degp[0, :N, 0:1] + degp[1, :N, 0:1]

    cbn = 1.0 / math.sqrt(1.0 + EPS)
    bp2 = bp.reshape(1, H).astype(f32)
    scales = [(cbn * g).reshape(1, H) for g in (G0, G1, G2)]
    shifts = [(b * cbn * g + bb).reshape(1, H)
              for (b, g, bb) in ((b0, G0, B0), (b1, G1, B1), (b2, G2, B2))]

    h, hwp = _tc_init(x, embrows, dcol, Wp, bp2, W0)

    parts = _get_sc_spmm()(src_p, dst_p, zH, hwp)
    h, hwp = _tc_layer(h, hwp, parts[0, :N], parts[1, :N], dcol,
                       R0, rb0.reshape(1, H), scales[0], shifts[0], W1)

    parts = _get_sc_spmm()(src_p, dst_p, zH, hwp)
    h, hwp = _tc_layer(h, hwp, parts[0, :N], parts[1, :N], dcol,
                       R1, rb1.reshape(1, H), scales[1], shifts[1], W2)

    parts = _get_sc_spmm()(src_p, dst_p, zH, hwp)
    h = _tc_last(h, hwp, parts[0, :N], parts[1, :N], dcol,
                 R2, rb2.reshape(1, H), scales[2], shifts[2])

    batch2d = batch.astype(jnp.int32).reshape(N, 1)
    return _tc_pool(h, batch2d, Wout, bout.reshape(1, OUT),
                    ln_g.reshape(1, OUT), ln_b.reshape(1, OUT))


# speedup vs baseline: 3.9650x; 1.1625x over previous
import functools
import math

import jax
import jax.numpy as jnp
from jax import lax
from jax.experimental import pallas as pl
from jax.experimental.pallas import tpu as pltpu
from jax.experimental.pallas import tpu_sc as plsc

N = 10000
E = 640000
D_IN = 128
H = 128
OUT = 768
VOCAB = 1000
G = 16
EPS = 1e-5

NC = 2
NS = 16
NW = NC * NS
CH = 128

E_PAD = 663552
EPW = E_PAD // NW
NCHUNK = EPW // CH

NID_PAD = 12288
ROWS_PER_W = NID_PAD // NW

ACC_ROWS = 10112
RPT = ACC_ROWS // NS
DUMMY = N

def _sc_mesh():
    return plsc.VectorSubcoreMesh(
        core_axis_name="c", subcore_axis_name="s",
        num_cores=NC, num_subcores=NS)


@functools.cache
def _get_sc_pre():
    return functools.partial(
        pl.kernel,
        out_type=(
            jax.ShapeDtypeStruct((NID_PAD, H), jnp.float32),
            jax.ShapeDtypeStruct((NC, ACC_ROWS, H), jnp.float32),
        ),
        mesh=_sc_mesh(),
        scratch_types=[
            pltpu.VMEM((CH,), jnp.int32),
            pltpu.VMEM((CH,), jnp.int32),
            pltpu.VMEM((CH,), jnp.int32),
            pltpu.VMEM((CH, H), jnp.float32),
            pltpu.VMEM_SHARED((ACC_ROWS, H), jnp.float32),
            pltpu.SemaphoreType.DMA,
            pltpu.SemaphoreType.DMA,
            pltpu.SemaphoreType.DMA,
        ],
    )(_sc_pre_body)


def _sc_pre_body(ids_hbm, dst_hbm, oh_hbm, z_hbm, emb_hbm,
                 embrows_hbm, deg_hbm,
                 di0, di1, ei_v, buf_v, acc_s, egsem, ds0, ds1):
    didx = [di0, di1]
    dsem = [ds0, ds1]
    ci = lax.axis_index("c")
    si = lax.axis_index("s")
    w = si * NC + ci
    row0 = si * RPT

    pltpu.sync_copy(z_hbm, buf_v)
    for k in range(4):
        pltpu.sync_copy(buf_v, acc_s.at[pl.ds(row0 + k * CH, CH)])
    pltpu.sync_copy(buf_v.at[pl.ds(0, RPT - 4 * CH)],
                    acc_s.at[pl.ds(row0 + 4 * CH, RPT - 4 * CH)])

    gbase = w * ROWS_PER_W
    for k in range(ROWS_PER_W // CH):
        b = gbase + k * CH
        pltpu.sync_copy(ids_hbm.at[pl.ds(b, CH)], ei_v)
        pltpu.async_copy(emb_hbm.at[ei_v], buf_v, egsem).wait()
        pltpu.sync_copy(buf_v, embrows_hbm.at[pl.ds(b, CH)])

    pltpu.sync_copy(oh_hbm, buf_v)
    plsc.subcore_barrier()

    ebase = w * EPW
    pltpu.sync_copy(dst_hbm.at[pl.ds(ebase, CH)], didx[0])

    def body(c0, carry):
        for bb in range(2):
            c = c0 * 2 + bb
            pltpu.async_copy(buf_v, acc_s.at[didx[bb]], dsem[bb], add=True)

            @pl.when(c + 1 < NCHUNK)
            def _():
                bq = (bb + 1) % 2

                @pl.when(c >= 1)
                def _():
                    pltpu.make_async_copy(buf_v, acc_s.at[didx[bq]],
                                          dsem[bq]).wait()

                pltpu.sync_copy(dst_hbm.at[pl.ds(ebase + (c + 1) * CH, CH)],
                                didx[bq])
        return carry

    lax.fori_loop(0, NCHUNK // 2, body, 0)
    for b in ((NCHUNK - 2) % 2, (NCHUNK - 1) % 2):
        pltpu.make_async_copy(buf_v, acc_s.at[didx[b]], dsem[b]).wait()

    plsc.subcore_barrier()
    for k in range(4):
        r = row0 + k * CH
        pltpu.sync_copy(acc_s.at[pl.ds(r, CH)], deg_hbm.at[ci, pl.ds(r, CH)])
    r = row0 + 4 * CH
    pltpu.sync_copy(acc_s.at[pl.ds(r, RPT - 4 * CH)],
                    deg_hbm.at[ci, pl.ds(r, RPT - 4 * CH)])


@functools.cache
def _get_sc_spmm():
    return functools.partial(
        pl.kernel,
        out_type=jax.ShapeDtypeStruct((NC, ACC_ROWS, H), jnp.float32),
        mesh=_sc_mesh(),
        scratch_types=(
            [pltpu.VMEM((CH,), jnp.int32)] * 3 +
            [pltpu.VMEM((CH,), jnp.int32)] * 3 +
            [pltpu.VMEM((CH, H), jnp.float32)] * 3 +
            [pltpu.VMEM_SHARED((ACC_ROWS, H), jnp.float32)] +
            [pltpu.SemaphoreType.DMA] * 6
        ),
    )(_sc_spmm_body)


def _sc_spmm_body(src_hbm, dst_hbm, z_hbm, table_hbm, parts_hbm,
                  si0, si1, si2, di0, di1, di2,
                  r0, r1, r2, acc_s, g0, g1, g2, s0, s1, s2):
    sidx = [si0, si1, si2]
    didx = [di0, di1, di2]
    rows = [r0, r1, r2]
    gsem = [g0, g1, g2]
    ssem = [s0, s1, s2]
    ci = lax.axis_index("c")
    si = lax.axis_index("s")
    w = si * NC + ci
    row0 = si * RPT

    pltpu.sync_copy(z_hbm, rows[0])
    for k in range(4):
        pltpu.sync_copy(rows[0], acc_s.at[pl.ds(row0 + k * CH, CH)])
    pltpu.sync_copy(rows[0].at[pl.ds(0, RPT - 4 * CH)],
                    acc_s.at[pl.ds(row0 + 4 * CH, RPT - 4 * CH)])

    plsc.subcore_barrier()

    ebase = w * EPW

    def fetch(c, b):
        pltpu.sync_copy(src_hbm.at[pl.ds(ebase + c * CH, CH)], sidx[b])
        pltpu.sync_copy(dst_hbm.at[pl.ds(ebase + c * CH, CH)], didx[b])
        pltpu.async_copy(table_hbm.at[sidx[b]], rows[b], gsem[b])

    fetch(0, 0)
    fetch(1, 1)

    def body(c0, carry):
        for bb in range(3):
            c = c0 * 3 + bb
            pltpu.make_async_copy(table_hbm.at[sidx[bb]], rows[bb],
                                  gsem[bb]).wait()
            pltpu.async_copy(rows[bb], acc_s.at[didx[bb]], ssem[bb], add=True)

            @pl.when(c + 2 < NCHUNK)
            def _():
                bq = (bb + 2) % 3

                @pl.when(c >= 1)
                def _():
                    pltpu.make_async_copy(rows[bq], acc_s.at[didx[bq]],
                                          ssem[bq]).wait()

                fetch(c + 2, bq)
        return carry

    lax.fori_loop(0, NCHUNK // 3, body, 0)
    for b in ((NCHUNK - 3) % 3, (NCHUNK - 2) % 3, (NCHUNK - 1) % 3):
        pltpu.make_async_copy(rows[b], acc_s.at[didx[b]], ssem[b]).wait()

    plsc.subcore_barrier()
    for k in range(4):
        r = row0 + k * CH
        pltpu.sync_copy(acc_s.at[pl.ds(r, CH)], parts_hbm.at[ci, pl.ds(r, CH)])
    r = row0 + 4 * CH
    pltpu.sync_copy(acc_s.at[pl.ds(r, RPT - 4 * CH)],
                    parts_hbm.at[ci, pl.ds(r, RPT - 4 * CH)])


_BLK = 1000


def _dot(a, b):
    return lax.dot_general(a, b, (((1,), (0,)), ((), ())),
                           precision=lax.Precision.HIGHEST,
                           preferred_element_type=jnp.float32)


def _dis_of(d_r):
    return lax.rsqrt(1.0 + d_r[...])


def _tc_init_body(x_r, er_r, d_r, wp_r, bp_r, w0_r, h_r, hwp_r):
    dis = _dis_of(d_r)
    h = _dot(x_r[...], wp_r[...]) + bp_r[...] + er_r[...]
    h_r[...] = h
    hwp_r[...] = _dot(h, w0_r[...]) * dis


def _tc_layer_body(h_r, hwp_r, p0_r, p1_r, d_r, r_r, rb_r, sc_r, sh_r,
                   wn_r, h2_r, hwp2_r):
    dis = _dis_of(d_r)
    s = p0_r[...] + p1_r[...] + hwp_r[...]
    z = jnp.maximum(dis * s * sc_r[...] + sh_r[...], 0.0)
    h2 = _dot(h_r[...], r_r[...]) + rb_r[...] + z
    h2_r[...] = h2
    hwp2_r[...] = _dot(h2, wn_r[...]) * dis


def _tc_last_body(h_r, hwp_r, p0_r, p1_r, d_r, r_r, rb_r, sc_r, sh_r,
                  h2_r):
    dis = _dis_of(d_r)
    s = p0_r[...] + p1_r[...] + hwp_r[...]
    z = jnp.maximum(dis * s * sc_r[...] + sh_r[...], 0.0)
    h2_r[...] = _dot(h_r[...], r_r[...]) + rb_r[...] + z


def _tc_pool_body(h_r, b_r, wout_r, bout_r, lng_r, lnb_r, out_r,
                  sums, maxs, cnts):
    i = pl.program_id(0)

    @pl.when(i == 0)
    def _init():
        sums[...] = jnp.zeros_like(sums)
        cnts[...] = jnp.zeros_like(cnts)
        maxs[...] = jnp.full_like(maxs, -jnp.inf)

    b = b_r[...]
    h = h_r[...]
    oh = (b == lax.broadcasted_iota(jnp.int32, (1, G), 1)).astype(jnp.float32)
    contract = (((0,), (0,)), ((), ()))
    sums[...] += lax.dot_general(oh, h, contract,
                                 precision=lax.Precision.HIGHEST,
                                 preferred_element_type=jnp.float32)
    cnts[...] += lax.dot_general(oh, jnp.ones_like(h), contract,
                                 precision=lax.Precision.HIGHEST,
                                 preferred_element_type=jnp.float32)
    blockmax = jnp.concatenate(
        [jnp.max(jnp.where(b == g, h, -jnp.inf), axis=0, keepdims=True)
         for g in range(G)], axis=0)
    maxs[...] = jnp.maximum(maxs[...], blockmax)

    @pl.when(i == pl.num_programs(0) - 1)
    def _fin():
        mean = sums[...] / jnp.maximum(cnts[...], 1.0)
        ge = jnp.concatenate([mean, maxs[...]], axis=1)
        y = _dot(ge, wout_r[...]) + bout_r[...]
        y = jnp.maximum(y, 0.0)
        mu = jnp.mean(y, axis=1, keepdims=True)
        var = jnp.mean((y - mu) ** 2, axis=1, keepdims=True)
        out_r[...] = (y - mu) * lax.rsqrt(var + EPS) * lng_r[...] + lnb_r[...]


def _row_spec(cols):
    return pl.BlockSpec((_BLK, cols), lambda i: (i, 0))


def _full_spec(rows, cols):
    return pl.BlockSpec((rows, cols), lambda i: (0, 0))


def _tc_init(x, embrows, dcol, Wp, bp, W0):
    return pl.pallas_call(
        _tc_init_body,
        grid=(N // _BLK,),
        in_specs=[_row_spec(H), _row_spec(H), _row_spec(1),
                  _full_spec(D_IN, H), _full_spec(1, H), _full_spec(H, H)],
        out_specs=[_row_spec(H), _row_spec(H)],
        out_shape=[jax.ShapeDtypeStruct((N, H), jnp.float32)] * 2,
    )(x, embrows, dcol, Wp, bp, W0)


def _tc_layer(h, hwp, p0, p1, dcol, R, rb, scale, shift, Wn):
    return pl.pallas_call(
        _tc_layer_body,
        grid=(N // _BLK,),
        in_specs=[_row_spec(H)] * 4 + [_row_spec(1)] +
                 [_full_spec(H, H), _full_spec(1, H), _full_spec(1, H),
                  _full_spec(1, H), _full_spec(H, H)],
        out_specs=[_row_spec(H), _row_spec(H)],
        out_shape=[jax.ShapeDtypeStruct((N, H), jnp.float32)] * 2,
    )(h, hwp, p0, p1, dcol, R, rb, scale, shift, Wn)


def _tc_last(h, hwp, p0, p1, dcol, R, rb, scale, shift):
    return pl.pallas_call(
        _tc_last_body,
        grid=(N // _BLK,),
        in_specs=[_row_spec(H)] * 4 + [_row_spec(1)] +
                 [_full_spec(H, H), _full_spec(1, H), _full_spec(1, H),
                  _full_spec(1, H)],
        out_specs=[_row_spec(H)],
        out_shape=[jax.ShapeDtypeStruct((N, H), jnp.float32)],
    )(h, hwp, p0, p1, dcol, R, rb, scale, shift)[0]


def _tc_pool(h, batch2d, Wout, bout, ln_g, ln_b):
    return pl.pallas_call(
        _tc_pool_body,
        grid=(N // _BLK,),
        in_specs=[_row_spec(H), _row_spec(1),
                  _full_spec(2 * H, OUT), _full_spec(1, OUT),
                  _full_spec(1, OUT), _full_spec(1, OUT)],
        out_specs=[_full_spec(G, OUT)],
        out_shape=[jax.ShapeDtypeStruct((G, OUT), jnp.float32)],
        scratch_shapes=[pltpu.VMEM((G, H), jnp.float32),
                        pltpu.VMEM((G, H), jnp.float32),
                        pltpu.VMEM((G, H), jnp.float32)],
    )(h, batch2d, Wout, bout, ln_g, ln_b)[0]


def kernel(x, node_ids, edge_index, batch, emb, Wp, bp,
           W0, b0, G0, B0, R0, rb0,
           W1, b1, G1, B1, R1, rb1,
           W2, b2, G2, B2, R2, rb2,
           Wout, bout, ln_g, ln_b):
    f32 = jnp.float32
    src = edge_index[0].astype(jnp.int32)
    dst = edge_index[1].astype(jnp.int32)
    pad_i = jnp.arange(E_PAD - E, dtype=jnp.int32)
    src_p = jnp.concatenate([src, pad_i % N])
    dst_p = jnp.concatenate([dst, DUMMY + pad_i % (ACC_ROWS - N)])
    ids_p = jnp.concatenate(
        [node_ids.astype(jnp.int32), jnp.zeros((NID_PAD - N,), jnp.int32)])

    ohH = jnp.concatenate(
        [jnp.ones((CH, 1), f32), jnp.zeros((CH, H - 1), f32)], axis=1)
    zH = jnp.zeros((CH, H), f32)

    embrows, degp = _get_sc_pre()(ids_p, dst_p, ohH, zH, emb)
    embrows = embrows[:N]
    dcol = degp[0, :N, 0:1] + degp[1, :N, 0:1]

    cbn = 1.0 / math.sqrt(1.0 + EPS)
    bp2 = bp.reshape(1, H).astype(f32)
    scales = [(cbn * g).reshape(1, H) for g in (G0, G1, G2)]
    shifts = [(b * cbn * g + bb).reshape(1, H)
              for (b, g, bb) in ((b0, G0, B0), (b1, G1, B1), (b2, G2, B2))]

    h, hwp = _tc_init(x, embrows, dcol, Wp, bp2, W0)

    parts = _get_sc_spmm()(src_p, dst_p, zH, hwp)
    h, hwp = _tc_layer(h, hwp, parts[0, :N], parts[1, :N], dcol,
                       R0, rb0.reshape(1, H), scales[0], shifts[0], W1)

    parts = _get_sc_spmm()(src_p, dst_p, zH, hwp)
    h, hwp = _tc_layer(h, hwp, parts[0, :N], parts[1, :N], dcol,
                       R1, rb1.reshape(1, H), scales[1], shifts[1], W2)

    parts = _get_sc_spmm()(src_p, dst_p, zH, hwp)
    h = _tc_last(h, hwp, parts[0, :N], parts[1, :N], dcol,
                 R2, rb2.reshape(1, H), scales[2], shifts[2])

    batch2d = batch.astype(jnp.int32).reshape(N, 1)
    return _tc_pool(h, batch2d, Wout, bout.reshape(1, OUT),
                    ln_g.reshape(1, OUT), ln_b.reshape(1, OUT))


# speedup vs baseline: 4.2732x; 1.0777x over previous
import functools
import math

import jax
import jax.numpy as jnp
from jax import lax
from jax.experimental import pallas as pl
from jax.experimental.pallas import tpu as pltpu
from jax.experimental.pallas import tpu_sc as plsc

N = 10000
E = 640000
D_IN = 128
H = 128
OUT = 768
VOCAB = 1000
G = 16
EPS = 1e-5

NC = 2
NS = 16
NW = NC * NS
CH = 128

E_PAD = 663552
EPW = E_PAD // NW
NCHUNK = EPW // CH

NID_PAD = 12288
ROWS_PER_W = NID_PAD // NW

ACC_ROWS = 10112
RPT = ACC_ROWS // NS
DUMMY = N

def _sc_mesh():
    return plsc.VectorSubcoreMesh(
        core_axis_name="c", subcore_axis_name="s",
        num_cores=NC, num_subcores=NS)


@functools.cache
def _get_sc_pre():
    return functools.partial(
        pl.kernel,
        out_type=(
            jax.ShapeDtypeStruct((NID_PAD, H), jnp.float32),
            jax.ShapeDtypeStruct((NC, ACC_ROWS, H), jnp.float32),
        ),
        mesh=_sc_mesh(),
        scratch_types=[
            pltpu.VMEM((CH,), jnp.int32),
            pltpu.VMEM((CH,), jnp.int32),
            pltpu.VMEM((CH,), jnp.int32),
            pltpu.VMEM((CH, H), jnp.float32),
            pltpu.VMEM_SHARED((ACC_ROWS, H), jnp.float32),
            pltpu.SemaphoreType.DMA,
            pltpu.SemaphoreType.DMA,
            pltpu.SemaphoreType.DMA,
        ],
    )(_sc_pre_body)


def _sc_pre_body(ids_hbm, dst_hbm, oh_hbm, z_hbm, emb_hbm,
                 embrows_hbm, deg_hbm,
                 di0, di1, ei_v, buf_v, acc_s, egsem, ds0, ds1):
    didx = [di0, di1]
    dsem = [ds0, ds1]
    ci = lax.axis_index("c")
    si = lax.axis_index("s")
    w = si * NC + ci
    row0 = si * RPT

    pltpu.sync_copy(z_hbm, buf_v)
    for k in range(4):
        pltpu.sync_copy(buf_v, acc_s.at[pl.ds(row0 + k * CH, CH)])
    pltpu.sync_copy(buf_v.at[pl.ds(0, RPT - 4 * CH)],
                    acc_s.at[pl.ds(row0 + 4 * CH, RPT - 4 * CH)])

    gbase = w * ROWS_PER_W
    for k in range(ROWS_PER_W // CH):
        b = gbase + k * CH
        pltpu.sync_copy(ids_hbm.at[pl.ds(b, CH)], ei_v)
        pltpu.async_copy(emb_hbm.at[ei_v], buf_v, egsem).wait()
        pltpu.sync_copy(buf_v, embrows_hbm.at[pl.ds(b, CH)])

    pltpu.sync_copy(oh_hbm, buf_v)
    plsc.subcore_barrier()

    ebase = w * EPW
    pltpu.sync_copy(dst_hbm.at[pl.ds(ebase, CH)], didx[0])

    def body(c0, carry):
        for bb in range(2):
            c = c0 * 2 + bb
            pltpu.async_copy(buf_v, acc_s.at[didx[bb]], dsem[bb], add=True)

            @pl.when(c + 1 < NCHUNK)
            def _():
                bq = (bb + 1) % 2

                @pl.when(c >= 1)
                def _():
                    pltpu.make_async_copy(buf_v, acc_s.at[didx[bq]],
                                          dsem[bq]).wait()

                pltpu.sync_copy(dst_hbm.at[pl.ds(ebase + (c + 1) * CH, CH)],
                                didx[bq])
        return carry

    lax.fori_loop(0, NCHUNK // 2, body, 0)
    for b in ((NCHUNK - 2) % 2, (NCHUNK - 1) % 2):
        pltpu.make_async_copy(buf_v, acc_s.at[didx[b]], dsem[b]).wait()

    plsc.subcore_barrier()
    for k in range(4):
        r = row0 + k * CH
        pltpu.sync_copy(acc_s.at[pl.ds(r, CH)], deg_hbm.at[ci, pl.ds(r, CH)])
    r = row0 + 4 * CH
    pltpu.sync_copy(acc_s.at[pl.ds(r, RPT - 4 * CH)],
                    deg_hbm.at[ci, pl.ds(r, RPT - 4 * CH)])


@functools.cache
def _get_sc_spmm():
    return functools.partial(
        pl.kernel,
        out_type=jax.ShapeDtypeStruct((NC, ACC_ROWS, H), jnp.float32),
        mesh=_sc_mesh(),
        scratch_types=(
            [pltpu.VMEM((CH,), jnp.int32)] * 3 +
            [pltpu.VMEM((CH,), jnp.int32)] * 3 +
            [pltpu.VMEM((CH, H), jnp.float32)] * 3 +
            [pltpu.VMEM_SHARED((ACC_ROWS, H), jnp.float32)] +
            [pltpu.SemaphoreType.DMA] * 12
        ),
    )(_sc_spmm_body)


def _sc_spmm_body(src_hbm, dst_hbm, z_hbm, table_hbm, parts_hbm, *refs):
    sidx = list(refs[0:3])
    didx = list(refs[3:6])
    rows = list(refs[6:9])
    acc_s = refs[9]
    isem = list(refs[10:13])
    jsem = list(refs[13:16])
    gsem = list(refs[16:19])
    ssem = list(refs[19:22])
    ci = lax.axis_index("c")
    si = lax.axis_index("s")
    w = si * NC + ci
    row0 = si * RPT

    pltpu.sync_copy(z_hbm, rows[0])
    for k in range(4):
        pltpu.sync_copy(rows[0], acc_s.at[pl.ds(row0 + k * CH, CH)])
    pltpu.sync_copy(rows[0].at[pl.ds(0, RPT - 4 * CH)],
                    acc_s.at[pl.ds(row0 + 4 * CH, RPT - 4 * CH)])

    plsc.subcore_barrier()

    ebase = w * EPW

    def fetch(c, b):
        da = pltpu.async_copy(src_hbm.at[pl.ds(ebase + c * CH, CH)], sidx[b],
                              isem[b])
        db = pltpu.async_copy(dst_hbm.at[pl.ds(ebase + c * CH, CH)], didx[b],
                              jsem[b])
        da.wait()
        db.wait()
        pltpu.async_copy(table_hbm.at[sidx[b]], rows[b], gsem[b])

    fetch(0, 0)
    fetch(1, 1)

    def body(c0, carry):
        for bb in range(3):
            c = c0 * 3 + bb
            pltpu.make_async_copy(table_hbm.at[sidx[bb]], rows[bb],
                                  gsem[bb]).wait()
            pltpu.async_copy(rows[bb], acc_s.at[didx[bb]], ssem[bb], add=True)

            @pl.when(c + 2 < NCHUNK)
            def _():
                bq = (bb + 2) % 3

                @pl.when(c >= 1)
                def _():
                    pltpu.make_async_copy(rows[bq], acc_s.at[didx[bq]],
                                          ssem[bq]).wait()

                fetch(c + 2, bq)
        return carry

    lax.fori_loop(0, NCHUNK // 3, body, 0)
    for b in ((NCHUNK - 3) % 3, (NCHUNK - 2) % 3, (NCHUNK - 1) % 3):
        pltpu.make_async_copy(rows[b], acc_s.at[didx[b]], ssem[b]).wait()

    plsc.subcore_barrier()
    for k in range(4):
        r = row0 + k * CH
        pltpu.sync_copy(acc_s.at[pl.ds(r, CH)], parts_hbm.at[ci, pl.ds(r, CH)])
    r = row0 + 4 * CH
    pltpu.sync_copy(acc_s.at[pl.ds(r, RPT - 4 * CH)],
                    parts_hbm.at[ci, pl.ds(r, RPT - 4 * CH)])


_BLK = 1000


def _dot(a, b):
    return lax.dot_general(a, b, (((1,), (0,)), ((), ())),
                           precision=lax.Precision.HIGHEST,
                           preferred_element_type=jnp.float32)


def _dis_of(d_r):
    return lax.rsqrt(1.0 + d_r[...])


def _tc_init_body(x_r, er_r, d_r, wp_r, bp_r, w0_r, h_r, hwp_r):
    dis = _dis_of(d_r)
    h = _dot(x_r[...], wp_r[...]) + bp_r[...] + er_r[...]
    h_r[...] = h
    hwp_r[...] = _dot(h, w0_r[...]) * dis


def _tc_layer_body(h_r, hwp_r, p0_r, p1_r, d_r, r_r, rb_r, sc_r, sh_r,
                   wn_r, h2_r, hwp2_r):
    dis = _dis_of(d_r)
    s = p0_r[...] + p1_r[...] + hwp_r[...]
    z = jnp.maximum(dis * s * sc_r[...] + sh_r[...], 0.0)
    h2 = _dot(h_r[...], r_r[...]) + rb_r[...] + z
    h2_r[...] = h2
    hwp2_r[...] = _dot(h2, wn_r[...]) * dis


def _tc_last_body(h_r, hwp_r, p0_r, p1_r, d_r, r_r, rb_r, sc_r, sh_r,
                  h2_r):
    dis = _dis_of(d_r)
    s = p0_r[...] + p1_r[...] + hwp_r[...]
    z = jnp.maximum(dis * s * sc_r[...] + sh_r[...], 0.0)
    h2_r[...] = _dot(h_r[...], r_r[...]) + rb_r[...] + z


def _tc_pool_body(h_r, b_r, wout_r, bout_r, lng_r, lnb_r, out_r,
                  sums, maxs, cnts):
    i = pl.program_id(0)

    @pl.when(i == 0)
    def _init():
        sums[...] = jnp.zeros_like(sums)
        cnts[...] = jnp.zeros_like(cnts)
        maxs[...] = jnp.full_like(maxs, -jnp.inf)

    b = b_r[...]
    h = h_r[...]
    oh = (b == lax.broadcasted_iota(jnp.int32, (1, G), 1)).astype(jnp.float32)
    contract = (((0,), (0,)), ((), ()))
    sums[...] += lax.dot_general(oh, h, contract,
                                 precision=lax.Precision.HIGHEST,
                                 preferred_element_type=jnp.float32)
    cnts[...] += lax.dot_general(oh, jnp.ones_like(h), contract,
                                 precision=lax.Precision.HIGHEST,
                                 preferred_element_type=jnp.float32)
    blockmax = jnp.concatenate(
        [jnp.max(jnp.where(b == g, h, -jnp.inf), axis=0, keepdims=True)
         for g in range(G)], axis=0)
    maxs[...] = jnp.maximum(maxs[...], blockmax)

    @pl.when(i == pl.num_programs(0) - 1)
    def _fin():
        mean = sums[...] / jnp.maximum(cnts[...], 1.0)
        ge = jnp.concatenate([mean, maxs[...]], axis=1)
        y = _dot(ge, wout_r[...]) + bout_r[...]
        y = jnp.maximum(y, 0.0)
        mu = jnp.mean(y, axis=1, keepdims=True)
        var = jnp.mean((y - mu) ** 2, axis=1, keepdims=True)
        out_r[...] = (y - mu) * lax.rsqrt(var + EPS) * lng_r[...] + lnb_r[...]


def _row_spec(cols):
    return pl.BlockSpec((_BLK, cols), lambda i: (i, 0))


def _full_spec(rows, cols):
    return pl.BlockSpec((rows, cols), lambda i: (0, 0))


def _tc_init(x, embrows, dcol, Wp, bp, W0):
    return pl.pallas_call(
        _tc_init_body,
        grid=(N // _BLK,),
        in_specs=[_row_spec(H), _row_spec(H), _row_spec(1),
                  _full_spec(D_IN, H), _full_spec(1, H), _full_spec(H, H)],
        out_specs=[_row_spec(H), _row_spec(H)],
        out_shape=[jax.ShapeDtypeStruct((N, H), jnp.float32)] * 2,
    )(x, embrows, dcol, Wp, bp, W0)


def _tc_layer(h, hwp, p0, p1, dcol, R, rb, scale, shift, Wn):
    return pl.pallas_call(
        _tc_layer_body,
        grid=(N // _BLK,),
        in_specs=[_row_spec(H)] * 4 + [_row_spec(1)] +
                 [_full_spec(H, H), _full_spec(1, H), _full_spec(1, H),
                  _full_spec(1, H), _full_spec(H, H)],
        out_specs=[_row_spec(H), _row_spec(H)],
        out_shape=[jax.ShapeDtypeStruct((N, H), jnp.float32)] * 2,
    )(h, hwp, p0, p1, dcol, R, rb, scale, shift, Wn)


def _tc_last(h, hwp, p0, p1, dcol, R, rb, scale, shift):
    return pl.pallas_call(
        _tc_last_body,
        grid=(N // _BLK,),
        in_specs=[_row_spec(H)] * 4 + [_row_spec(1)] +
                 [_full_spec(H, H), _full_spec(1, H), _full_spec(1, H),
                  _full_spec(1, H)],
        out_specs=[_row_spec(H)],
        out_shape=[jax.ShapeDtypeStruct((N, H), jnp.float32)],
    )(h, hwp, p0, p1, dcol, R, rb, scale, shift)[0]


def _tc_pool(h, batch2d, Wout, bout, ln_g, ln_b):
    return pl.pallas_call(
        _tc_pool_body,
        grid=(N // _BLK,),
        in_specs=[_row_spec(H), _row_spec(1),
                  _full_spec(2 * H, OUT), _full_spec(1, OUT),
                  _full_spec(1, OUT), _full_spec(1, OUT)],
        out_specs=[_full_spec(G, OUT)],
        out_shape=[jax.ShapeDtypeStruct((G, OUT), jnp.float32)],
        scratch_shapes=[pltpu.VMEM((G, H), jnp.float32),
                        pltpu.VMEM((G, H), jnp.float32),
                        pltpu.VMEM((G, H), jnp.float32)],
    )(h, batch2d, Wout, bout, ln_g, ln_b)[0]


def kernel(x, node_ids, edge_index, batch, emb, Wp, bp,
           W0, b0, G0, B0, R0, rb0,
           W1, b1, G1, B1, R1, rb1,
           W2, b2, G2, B2, R2, rb2,
           Wout, bout, ln_g, ln_b):
    f32 = jnp.float32
    src = edge_index[0].astype(jnp.int32)
    dst = edge_index[1].astype(jnp.int32)
    pad_i = jnp.arange(E_PAD - E, dtype=jnp.int32)
    src_p = jnp.concatenate([src, pad_i % N])
    dst_p = jnp.concatenate([dst, DUMMY + pad_i % (ACC_ROWS - N)])
    ids_p = jnp.concatenate(
        [node_ids.astype(jnp.int32), jnp.zeros((NID_PAD - N,), jnp.int32)])

    ohH = jnp.concatenate(
        [jnp.ones((CH, 1), f32), jnp.zeros((CH, H - 1), f32)], axis=1)
    zH = jnp.zeros((CH, H), f32)

    embrows, degp = _get_sc_pre()(ids_p, dst_p, ohH, zH, emb)
    embrows = embrows[:N]
    dcol = degp[0, :N, 0:1] + degp[1, :N, 0:1]

    cbn = 1.0 / math.sqrt(1.0 + EPS)
    bp2 = bp.reshape(1, H).astype(f32)
    scales = [(cbn * g).reshape(1, H) for g in (G0, G1, G2)]
    shifts = [(b * cbn * g + bb).reshape(1, H)
              for (b, g, bb) in ((b0, G0, B0), (b1, G1, B1), (b2, G2, B2))]

    h, hwp = _tc_init(x, embrows, dcol, Wp, bp2, W0)

    parts = _get_sc_spmm()(src_p, dst_p, zH, hwp)
    h, hwp = _tc_layer(h, hwp, parts[0, :N], parts[1, :N], dcol,
                       R0, rb0.reshape(1, H), scales[0], shifts[0], W1)

    parts = _get_sc_spmm()(src_p, dst_p, zH, hwp)
    h, hwp = _tc_layer(h, hwp, parts[0, :N], parts[1, :N], dcol,
                       R1, rb1.reshape(1, H), scales[1], shifts[1], W2)

    parts = _get_sc_spmm()(src_p, dst_p, zH, hwp)
    h = _tc_last(h, hwp, parts[0, :N], parts[1, :N], dcol,
                 R2, rb2.reshape(1, H), scales[2], shifts[2])

    batch2d = batch.astype(jnp.int32).reshape(N, 1)
    return _tc_pool(h, batch2d, Wout, bout.reshape(1, OUT),
                    ln_g.reshape(1, OUT), ln_b.reshape(1, OUT))


# speedup vs baseline: 4.3191x; 1.0108x over previous
import functools
import math

import jax
import jax.numpy as jnp
from jax import lax
from jax.experimental import pallas as pl
from jax.experimental.pallas import tpu as pltpu
from jax.experimental.pallas import tpu_sc as plsc

N = 10000
E = 640000
D_IN = 128
H = 128
OUT = 768
VOCAB = 1000
G = 16
EPS = 1e-5

NC = 2
NS = 16
NW = NC * NS
CH = 128

E_PAD = 663552
EPW = E_PAD // NW
NCHUNK = EPW // CH

NID_PAD = 12288
ROWS_PER_W = NID_PAD // NW

ACC_ROWS = 10112
RPT = ACC_ROWS // NS
DUMMY = N

def _sc_mesh():
    return plsc.VectorSubcoreMesh(
        core_axis_name="c", subcore_axis_name="s",
        num_cores=NC, num_subcores=NS)


@functools.cache
def _get_sc_pre():
    return functools.partial(
        pl.kernel,
        out_type=(
            jax.ShapeDtypeStruct((NID_PAD, H), jnp.float32),
            jax.ShapeDtypeStruct((NC, ACC_ROWS, H), jnp.float32),
        ),
        mesh=_sc_mesh(),
        scratch_types=[
            pltpu.VMEM((CH,), jnp.int32),
            pltpu.VMEM((CH,), jnp.int32),
            pltpu.VMEM((CH,), jnp.int32),
            pltpu.VMEM((CH, H), jnp.float32),
            pltpu.VMEM_SHARED((ACC_ROWS, H), jnp.float32),
            pltpu.SemaphoreType.DMA,
            pltpu.SemaphoreType.DMA,
            pltpu.SemaphoreType.DMA,
        ],
    )(_sc_pre_body)


def _sc_pre_body(ids_hbm, dst_hbm, oh_hbm, z_hbm, emb_hbm,
                 embrows_hbm, deg_hbm,
                 di0, di1, ei_v, buf_v, acc_s, egsem, ds0, ds1):
    didx = [di0, di1]
    dsem = [ds0, ds1]
    ci = lax.axis_index("c")
    si = lax.axis_index("s")
    w = si * NC + ci
    row0 = si * RPT

    pltpu.sync_copy(z_hbm, buf_v)
    for k in range(4):
        pltpu.sync_copy(buf_v, acc_s.at[pl.ds(row0 + k * CH, CH)])
    pltpu.sync_copy(buf_v.at[pl.ds(0, RPT - 4 * CH)],
                    acc_s.at[pl.ds(row0 + 4 * CH, RPT - 4 * CH)])

    gbase = w * ROWS_PER_W
    for k in range(ROWS_PER_W // CH):
        b = gbase + k * CH
        pltpu.sync_copy(ids_hbm.at[pl.ds(b, CH)], ei_v)
        pltpu.async_copy(emb_hbm.at[ei_v], buf_v, egsem).wait()
        pltpu.sync_copy(buf_v, embrows_hbm.at[pl.ds(b, CH)])

    pltpu.sync_copy(oh_hbm, buf_v)
    plsc.subcore_barrier()

    ebase = w * EPW
    pltpu.sync_copy(dst_hbm.at[pl.ds(ebase, CH)], didx[0])

    def body(c0, carry):
        for bb in range(2):
            c = c0 * 2 + bb
            pltpu.async_copy(buf_v, acc_s.at[didx[bb]], dsem[bb], add=True)

            @pl.when(c + 1 < NCHUNK)
            def _():
                bq = (bb + 1) % 2

                @pl.when(c >= 1)
                def _():
                    pltpu.make_async_copy(buf_v, acc_s.at[didx[bq]],
                                          dsem[bq]).wait()

                pltpu.sync_copy(dst_hbm.at[pl.ds(ebase + (c + 1) * CH, CH)],
                                didx[bq])
        return carry

    lax.fori_loop(0, NCHUNK // 2, body, 0)
    for b in ((NCHUNK - 2) % 2, (NCHUNK - 1) % 2):
        pltpu.make_async_copy(buf_v, acc_s.at[didx[b]], dsem[b]).wait()

    plsc.subcore_barrier()
    for k in range(4):
        r = row0 + k * CH
        pltpu.sync_copy(acc_s.at[pl.ds(r, CH)], deg_hbm.at[ci, pl.ds(r, CH)])
    r = row0 + 4 * CH
    pltpu.sync_copy(acc_s.at[pl.ds(r, RPT - 4 * CH)],
                    deg_hbm.at[ci, pl.ds(r, RPT - 4 * CH)])


@functools.cache
def _get_sc_spmm():
    return functools.partial(
        pl.kernel,
        out_type=jax.ShapeDtypeStruct((NC, ACC_ROWS, H), jnp.float32),
        mesh=_sc_mesh(),
        scratch_types=(
            [pltpu.VMEM((CH,), jnp.int32)] * 3 +
            [pltpu.VMEM((CH,), jnp.int32)] * 3 +
            [pltpu.VMEM((CH, H), jnp.float32)] * 3 +
            [pltpu.VMEM_SHARED((ACC_ROWS, H), jnp.float32)] +
            [pltpu.SemaphoreType.DMA] * 12
        ),
    )(_sc_spmm_body)


def _sc_spmm_body(src_hbm, dst_hbm, z_hbm, table_hbm, parts_hbm, *refs):
    sidx = list(refs[0:3])
    didx = list(refs[3:6])
    rows = list(refs[6:9])
    acc_s = refs[9]
    isem = list(refs[10:13])
    jsem = list(refs[13:16])
    gsem = list(refs[16:19])
    ssem = list(refs[19:22])
    ci = lax.axis_index("c")
    si = lax.axis_index("s")
    w = si * NC + ci
    row0 = si * RPT

    pltpu.sync_copy(z_hbm, rows[0])
    for k in range(4):
        pltpu.sync_copy(rows[0], acc_s.at[pl.ds(row0 + k * CH, CH)])
    pltpu.sync_copy(rows[0].at[pl.ds(0, RPT - 4 * CH)],
                    acc_s.at[pl.ds(row0 + 4 * CH, RPT - 4 * CH)])

    plsc.subcore_barrier()

    ebase = w * EPW

    def fetch(c, b):
        da = pltpu.async_copy(src_hbm.at[pl.ds(ebase + c * CH, CH)], sidx[b],
                              isem[b])
        db = pltpu.async_copy(dst_hbm.at[pl.ds(ebase + c * CH, CH)], didx[b],
                              jsem[b])
        da.wait()
        db.wait()
        pltpu.async_copy(table_hbm.at[sidx[b]], rows[b], gsem[b])

    fetch(0, 0)
    fetch(1, 1)

    def body(c0, carry):
        for bb in range(3):
            c = c0 * 3 + bb
            pltpu.make_async_copy(table_hbm.at[sidx[bb]], rows[bb],
                                  gsem[bb]).wait()
            pltpu.async_copy(rows[bb], acc_s.at[didx[bb]], ssem[bb], add=True)

            @pl.when(c + 2 < NCHUNK)
            def _():
                bq = (bb + 2) % 3

                @pl.when(c >= 1)
                def _():
                    pltpu.make_async_copy(rows[bq], acc_s.at[didx[bq]],
                                          ssem[bq]).wait()

                fetch(c + 2, bq)
        return carry

    lax.fori_loop(0, NCHUNK // 3, body, 0)
    for b in ((NCHUNK - 3) % 3, (NCHUNK - 2) % 3, (NCHUNK - 1) % 3):
        pltpu.make_async_copy(rows[b], acc_s.at[didx[b]], ssem[b]).wait()

    plsc.subcore_barrier()
    for k in range(4):
        r = row0 + k * CH
        pltpu.sync_copy(acc_s.at[pl.ds(r, CH)], parts_hbm.at[ci, pl.ds(r, CH)])
    r = row0 + 4 * CH
    pltpu.sync_copy(acc_s.at[pl.ds(r, RPT - 4 * CH)],
                    parts_hbm.at[ci, pl.ds(r, RPT - 4 * CH)])


_BLK = 1000


def _dot(a, b):
    return lax.dot_general(a, b, (((1,), (0,)), ((), ())),
                           precision=lax.Precision.HIGHEST,
                           preferred_element_type=jnp.float32)


def _dis_of(d_r):
    return lax.rsqrt(1.0 + d_r[...])


def _tc_init_body(x_r, er_r, d_r, wp_r, bp_r, w0_r, h_r, hwp_r):
    dis = _dis_of(d_r)
    h = _dot(x_r[...], wp_r[...]) + bp_r[...] + er_r[...]
    h_r[...] = h
    hwp_r[...] = _dot(h, w0_r[...]) * dis


def _tc_layer_body(h_r, hwp_r, p0_r, p1_r, d_r, r_r, rb_r, sc_r, sh_r,
                   wn_r, h2_r, hwp2_r):
    dis = _dis_of(d_r)
    s = p0_r[...] + p1_r[...] + hwp_r[...]
    z = jnp.maximum(dis * s * sc_r[...] + sh_r[...], 0.0)
    h2 = _dot(h_r[...], r_r[...]) + rb_r[...] + z
    h2_r[...] = h2
    hwp2_r[...] = _dot(h2, wn_r[...]) * dis


def _tc_pool_body(h_r, hwp_r, p0_r, p1_r, d_r, r_r, rb_r, sc_r, sh_r,
                  b_r, wout_r, bout_r, lng_r, lnb_r, out_r,
                  sums, maxs, cnts):
    i = pl.program_id(0)

    @pl.when(i == 0)
    def _init():
        sums[...] = jnp.zeros_like(sums)
        cnts[...] = jnp.zeros_like(cnts)
        maxs[...] = jnp.full_like(maxs, -jnp.inf)

    dis = _dis_of(d_r)
    s = p0_r[...] + p1_r[...] + hwp_r[...]
    z = jnp.maximum(dis * s * sc_r[...] + sh_r[...], 0.0)
    h = _dot(h_r[...], r_r[...]) + rb_r[...] + z

    b = b_r[...]
    oh = (b == lax.broadcasted_iota(jnp.int32, (1, G), 1)).astype(jnp.float32)
    contract = (((0,), (0,)), ((), ()))
    sums[...] += lax.dot_general(oh, h, contract,
                                 precision=lax.Precision.HIGHEST,
                                 preferred_element_type=jnp.float32)
    cnts[...] += lax.dot_general(oh, jnp.ones_like(h), contract,
                                 precision=lax.Precision.HIGHEST,
                                 preferred_element_type=jnp.float32)
    blockmax = jnp.concatenate(
        [jnp.max(jnp.where(b == g, h, -jnp.inf), axis=0, keepdims=True)
         for g in range(G)], axis=0)
    maxs[...] = jnp.maximum(maxs[...], blockmax)

    @pl.when(i == pl.num_programs(0) - 1)
    def _fin():
        mean = sums[...] / jnp.maximum(cnts[...], 1.0)
        ge = jnp.concatenate([mean, maxs[...]], axis=1)
        y = _dot(ge, wout_r[...]) + bout_r[...]
        y = jnp.maximum(y, 0.0)
        mu = jnp.mean(y, axis=1, keepdims=True)
        var = jnp.mean((y - mu) ** 2, axis=1, keepdims=True)
        out_r[...] = (y - mu) * lax.rsqrt(var + EPS) * lng_r[...] + lnb_r[...]


def _row_spec(cols):
    return pl.BlockSpec((_BLK, cols), lambda i: (i, 0))


def _full_spec(rows, cols):
    return pl.BlockSpec((rows, cols), lambda i: (0, 0))


def _tc_init(x, embrows, dcol, Wp, bp, W0):
    return pl.pallas_call(
        _tc_init_body,
        grid=(N // _BLK,),
        in_specs=[_row_spec(H), _row_spec(H), _row_spec(1),
                  _full_spec(D_IN, H), _full_spec(1, H), _full_spec(H, H)],
        out_specs=[_row_spec(H), _row_spec(H)],
        out_shape=[jax.ShapeDtypeStruct((N, H), jnp.float32)] * 2,
    )(x, embrows, dcol, Wp, bp, W0)


def _tc_layer(h, hwp, p0, p1, dcol, R, rb, scale, shift, Wn):
    return pl.pallas_call(
        _tc_layer_body,
        grid=(N // _BLK,),
        in_specs=[_row_spec(H)] * 4 + [_row_spec(1)] +
                 [_full_spec(H, H), _full_spec(1, H), _full_spec(1, H),
                  _full_spec(1, H), _full_spec(H, H)],
        out_specs=[_row_spec(H), _row_spec(H)],
        out_shape=[jax.ShapeDtypeStruct((N, H), jnp.float32)] * 2,
    )(h, hwp, p0, p1, dcol, R, rb, scale, shift, Wn)


def _tc_pool(h, hwp, p0, p1, dcol, R, rb, scale, shift,
             batch2d, Wout, bout, ln_g, ln_b):
    return pl.pallas_call(
        _tc_pool_body,
        grid=(N // _BLK,),
        in_specs=[_row_spec(H)] * 4 + [_row_spec(1)] +
                 [_full_spec(H, H), _full_spec(1, H), _full_spec(1, H),
                  _full_spec(1, H)] +
                 [_row_spec(1),
                  _full_spec(2 * H, OUT), _full_spec(1, OUT),
                  _full_spec(1, OUT), _full_spec(1, OUT)],
        out_specs=[_full_spec(G, OUT)],
        out_shape=[jax.ShapeDtypeStruct((G, OUT), jnp.float32)],
        scratch_shapes=[pltpu.VMEM((G, H), jnp.float32),
                        pltpu.VMEM((G, H), jnp.float32),
                        pltpu.VMEM((G, H), jnp.float32)],
    )(h, hwp, p0, p1, dcol, R, rb, scale, shift,
      batch2d, Wout, bout, ln_g, ln_b)[0]


def kernel(x, node_ids, edge_index, batch, emb, Wp, bp,
           W0, b0, G0, B0, R0, rb0,
           W1, b1, G1, B1, R1, rb1,
           W2, b2, G2, B2, R2, rb2,
           Wout, bout, ln_g, ln_b):
    f32 = jnp.float32
    src = edge_index[0].astype(jnp.int32)
    dst = edge_index[1].astype(jnp.int32)
    pad_i = jnp.arange(E_PAD - E, dtype=jnp.int32)
    src_p = jnp.concatenate([src, pad_i % N])
    dst_p = jnp.concatenate([dst, DUMMY + pad_i % (ACC_ROWS - N)])
    ids_p = jnp.concatenate(
        [node_ids.astype(jnp.int32), jnp.zeros((NID_PAD - N,), jnp.int32)])

    ohH = jnp.concatenate(
        [jnp.ones((CH, 1), f32), jnp.zeros((CH, H - 1), f32)], axis=1)
    zH = jnp.zeros((CH, H), f32)

    embrows, degp = _get_sc_pre()(ids_p, dst_p, ohH, zH, emb)
    embrows = embrows[:N]
    dcol = degp[0, :N, 0:1] + degp[1, :N, 0:1]

    cbn = 1.0 / math.sqrt(1.0 + EPS)
    bp2 = bp.reshape(1, H).astype(f32)
    scales = [(cbn * g).reshape(1, H) for g in (G0, G1, G2)]
    shifts = [(b * cbn * g + bb).reshape(1, H)
              for (b, g, bb) in ((b0, G0, B0), (b1, G1, B1), (b2, G2, B2))]

    h, hwp = _tc_init(x, embrows, dcol, Wp, bp2, W0)

    parts = _get_sc_spmm()(src_p, dst_p, zH, hwp)
    h, hwp = _tc_layer(h, hwp, parts[0, :N], parts[1, :N], dcol,
                       R0, rb0.reshape(1, H), scales[0], shifts[0], W1)

    parts = _get_sc_spmm()(src_p, dst_p, zH, hwp)
    h, hwp = _tc_layer(h, hwp, parts[0, :N], parts[1, :N], dcol,
                       R1, rb1.reshape(1, H), scales[1], shifts[1], W2)

    parts = _get_sc_spmm()(src_p, dst_p, zH, hwp)
    batch2d = batch.astype(jnp.int32).reshape(N, 1)
    return _tc_pool(h, hwp, parts[0, :N], parts[1, :N], dcol,
                    R2, rb2.reshape(1, H), scales[2], shifts[2],
                    batch2d, Wout, bout.reshape(1, OUT),
                    ln_g.reshape(1, OUT), ln_b.reshape(1, OUT))


# speedup vs baseline: 4.5311x; 1.0491x over previous
import functools
import math

import jax
import jax.numpy as jnp
from jax import lax
from jax.experimental import pallas as pl
from jax.experimental.pallas import tpu as pltpu
from jax.experimental.pallas import tpu_sc as plsc

N = 10000
E = 640000
D_IN = 128
H = 128
OUT = 768
VOCAB = 1000
G = 16
EPS = 1e-5

NC = 2
NS = 16
NW = NC * NS
CH = 128

E_PAD = 663552
EPW = E_PAD // NW
NCHUNK = EPW // CH

NID_PAD = 12288
ROWS_PER_W = NID_PAD // NW

ACC_ROWS = 10112
RPT = ACC_ROWS // NS
DUMMY = N

def _sc_mesh():
    return plsc.VectorSubcoreMesh(
        core_axis_name="c", subcore_axis_name="s",
        num_cores=NC, num_subcores=NS)


@functools.cache
def _get_sc_pre():
    return functools.partial(
        pl.kernel,
        out_type=(
            jax.ShapeDtypeStruct((NID_PAD, H), jnp.float32),
            jax.ShapeDtypeStruct((NC, ACC_ROWS, H), jnp.float32),
        ),
        mesh=_sc_mesh(),
        scratch_types=[
            pltpu.VMEM((CH,), jnp.int32),
            pltpu.VMEM((CH,), jnp.int32),
            pltpu.VMEM((CH,), jnp.int32),
            pltpu.VMEM((CH, H), jnp.float32),
            pltpu.VMEM_SHARED((ACC_ROWS, H), jnp.float32),
            pltpu.SemaphoreType.DMA,
            pltpu.SemaphoreType.DMA,
            pltpu.SemaphoreType.DMA,
        ],
    )(_sc_pre_body)


def _sc_pre_body(ids_hbm, dst_hbm, oh_hbm, z_hbm, emb_hbm,
                 embrows_hbm, deg_hbm,
                 di0, di1, ei_v, buf_v, acc_s, egsem, ds0, ds1):
    didx = [di0, di1]
    dsem = [ds0, ds1]
    ci = lax.axis_index("c")
    si = lax.axis_index("s")
    w = si * NC + ci
    row0 = si * RPT

    pltpu.sync_copy(z_hbm, buf_v)
    for k in range(4):
        pltpu.sync_copy(buf_v, acc_s.at[pl.ds(row0 + k * CH, CH)])
    pltpu.sync_copy(buf_v.at[pl.ds(0, RPT - 4 * CH)],
                    acc_s.at[pl.ds(row0 + 4 * CH, RPT - 4 * CH)])

    gbase = w * ROWS_PER_W
    for k in range(ROWS_PER_W // CH):
        b = gbase + k * CH
        pltpu.sync_copy(ids_hbm.at[pl.ds(b, CH)], ei_v)
        pltpu.async_copy(emb_hbm.at[ei_v], buf_v, egsem).wait()
        pltpu.sync_copy(buf_v, embrows_hbm.at[pl.ds(b, CH)])

    pltpu.sync_copy(oh_hbm, buf_v)
    plsc.subcore_barrier()

    ebase = w * EPW
    pltpu.sync_copy(dst_hbm.at[pl.ds(ebase, CH)], didx[0])

    def body(c0, carry):
        for bb in range(2):
            c = c0 * 2 + bb
            pltpu.async_copy(buf_v, acc_s.at[didx[bb]], dsem[bb], add=True)

            @pl.when(c + 1 < NCHUNK)
            def _():
                bq = (bb + 1) % 2

                @pl.when(c >= 1)
                def _():
                    pltpu.make_async_copy(buf_v, acc_s.at[didx[bq]],
                                          dsem[bq]).wait()

                pltpu.sync_copy(dst_hbm.at[pl.ds(ebase + (c + 1) * CH, CH)],
                                didx[bq])
        return carry

    lax.fori_loop(0, NCHUNK // 2, body, 0)
    for b in ((NCHUNK - 2) % 2, (NCHUNK - 1) % 2):
        pltpu.make_async_copy(buf_v, acc_s.at[didx[b]], dsem[b]).wait()

    plsc.subcore_barrier()
    for k in range(4):
        r = row0 + k * CH
        pltpu.sync_copy(acc_s.at[pl.ds(r, CH)], deg_hbm.at[ci, pl.ds(r, CH)])
    r = row0 + 4 * CH
    pltpu.sync_copy(acc_s.at[pl.ds(r, RPT - 4 * CH)],
                    deg_hbm.at[ci, pl.ds(r, RPT - 4 * CH)])


@functools.cache
def _get_sc_spmm():
    return functools.partial(
        pl.kernel,
        out_type=jax.ShapeDtypeStruct((NC, ACC_ROWS, H), jnp.float32),
        mesh=_sc_mesh(),
        scratch_types=(
            [pltpu.VMEM((CH,), jnp.int32)] * 3 +
            [pltpu.VMEM((CH,), jnp.int32)] * 3 +
            [pltpu.VMEM((CH, H), jnp.float32)] * 3 +
            [pltpu.VMEM_SHARED((ACC_ROWS, H), jnp.float32)] +
            [pltpu.SemaphoreType.DMA] * 12
        ),
    )(_sc_spmm_body)


def _sc_spmm_body(src_hbm, dst_hbm, z_hbm, table_hbm, parts_hbm, *refs):
    sidx = list(refs[0:3])
    didx = list(refs[3:6])
    rows = list(refs[6:9])
    acc_s = refs[9]
    isem = list(refs[10:13])
    jsem = list(refs[13:16])
    gsem = list(refs[16:19])
    ssem = list(refs[19:22])
    ci = lax.axis_index("c")
    si = lax.axis_index("s")
    w = si * NC + ci
    row0 = si * RPT

    pltpu.sync_copy(z_hbm, rows[0])
    for k in range(4):
        pltpu.sync_copy(rows[0], acc_s.at[pl.ds(row0 + k * CH, CH)])
    pltpu.sync_copy(rows[0].at[pl.ds(0, RPT - 4 * CH)],
                    acc_s.at[pl.ds(row0 + 4 * CH, RPT - 4 * CH)])

    plsc.subcore_barrier()

    ebase = w * EPW

    def fetch(c, b):
        da = pltpu.async_copy(src_hbm.at[pl.ds(ebase + c * CH, CH)], sidx[b],
                              isem[b])
        db = pltpu.async_copy(dst_hbm.at[pl.ds(ebase + c * CH, CH)], didx[b],
                              jsem[b])
        da.wait()
        db.wait()
        pltpu.async_copy(table_hbm.at[sidx[b]], rows[b], gsem[b])

    fetch(0, 0)
    fetch(1, 1)

    def body(c0, carry):
        for bb in range(3):
            c = c0 * 3 + bb
            pltpu.make_async_copy(table_hbm.at[sidx[bb]], rows[bb],
                                  gsem[bb]).wait()
            pltpu.async_copy(rows[bb], acc_s.at[didx[bb]], ssem[bb], add=True)

            @pl.when(c + 2 < NCHUNK)
            def _():
                bq = (bb + 2) % 3

                @pl.when(c >= 1)
                def _():
                    pltpu.make_async_copy(rows[bq], acc_s.at[didx[bq]],
                                          ssem[bq]).wait()

                fetch(c + 2, bq)
        return carry

    lax.fori_loop(0, NCHUNK // 3, body, 0)
    for b in ((NCHUNK - 3) % 3, (NCHUNK - 2) % 3, (NCHUNK - 1) % 3):
        pltpu.make_async_copy(rows[b], acc_s.at[didx[b]], ssem[b]).wait()

    plsc.subcore_barrier()
    for k in range(4):
        r = row0 + k * CH
        pltpu.sync_copy(acc_s.at[pl.ds(r, CH)], parts_hbm.at[ci, pl.ds(r, CH)])
    r = row0 + 4 * CH
    pltpu.sync_copy(acc_s.at[pl.ds(r, RPT - 4 * CH)],
                    parts_hbm.at[ci, pl.ds(r, RPT - 4 * CH)])


_BLK = 2000


def _dot(a, b):
    return lax.dot_general(a, b, (((1,), (0,)), ((), ())),
                           precision=lax.Precision.HIGHEST,
                           preferred_element_type=jnp.float32)


def _dis_of(d_r):
    return lax.rsqrt(1.0 + d_r[...])


def _tc_proj_body(x_r, wp_r, bp_r, t_r):
    t_r[...] = _dot(x_r[...], wp_r[...]) + bp_r[...]


def _tc_init_body(t_r, er_r, d_r, w0_r, h_r, hwp_r):
    dis = _dis_of(d_r)
    h = t_r[...] + er_r[...]
    h_r[...] = h
    hwp_r[...] = _dot(h, w0_r[...]) * dis


def _tc_layer_body(h_r, hwp_r, p0_r, p1_r, d_r, r_r, rb_r, sc_r, sh_r,
                   wn_r, h2_r, hwp2_r):
    dis = _dis_of(d_r)
    s = p0_r[...] + p1_r[...] + hwp_r[...]
    z = jnp.maximum(dis * s * sc_r[...] + sh_r[...], 0.0)
    h2 = _dot(h_r[...], r_r[...]) + rb_r[...] + z
    h2_r[...] = h2
    hwp2_r[...] = _dot(h2, wn_r[...]) * dis


def _tc_pool_body(h_r, hwp_r, p0_r, p1_r, d_r, r_r, rb_r, sc_r, sh_r,
                  b_r, wout_r, bout_r, lng_r, lnb_r, out_r,
                  sums, maxs, cnts):
    i = pl.program_id(0)

    @pl.when(i == 0)
    def _init():
        sums[...] = jnp.zeros_like(sums)
        cnts[...] = jnp.zeros_like(cnts)
        maxs[...] = jnp.full_like(maxs, -jnp.inf)

    dis = _dis_of(d_r)
    s = p0_r[...] + p1_r[...] + hwp_r[...]
    z = jnp.maximum(dis * s * sc_r[...] + sh_r[...], 0.0)
    h = _dot(h_r[...], r_r[...]) + rb_r[...] + z

    b = b_r[...]
    oh = (b == lax.broadcasted_iota(jnp.int32, (1, G), 1)).astype(jnp.float32)
    contract = (((0,), (0,)), ((), ()))
    sums[...] += lax.dot_general(oh, h, contract,
                                 precision=lax.Precision.HIGHEST,
                                 preferred_element_type=jnp.float32)
    cnts[...] += lax.dot_general(oh, jnp.ones_like(h), contract,
                                 precision=lax.Precision.HIGHEST,
                                 preferred_element_type=jnp.float32)
    blockmax = jnp.concatenate(
        [jnp.max(jnp.where(b == g, h, -jnp.inf), axis=0, keepdims=True)
         for g in range(G)], axis=0)
    maxs[...] = jnp.maximum(maxs[...], blockmax)

    @pl.when(i == pl.num_programs(0) - 1)
    def _fin():
        mean = sums[...] / jnp.maximum(cnts[...], 1.0)
        ge = jnp.concatenate([mean, maxs[...]], axis=1)
        y = _dot(ge, wout_r[...]) + bout_r[...]
        y = jnp.maximum(y, 0.0)
        mu = jnp.mean(y, axis=1, keepdims=True)
        var = jnp.mean((y - mu) ** 2, axis=1, keepdims=True)
        out_r[...] = (y - mu) * lax.rsqrt(var + EPS) * lng_r[...] + lnb_r[...]


def _row_spec(cols):
    return pl.BlockSpec((_BLK, cols), lambda i: (i, 0))


def _full_spec(rows, cols):
    return pl.BlockSpec((rows, cols), lambda i: (0, 0))


def _tc_proj(x, Wp, bp):
    return pl.pallas_call(
        _tc_proj_body,
        grid=(N // _BLK,),
        in_specs=[_row_spec(H), _full_spec(D_IN, H), _full_spec(1, H)],
        out_specs=[_row_spec(H)],
        out_shape=[jax.ShapeDtypeStruct((N, H), jnp.float32)],
    )(x, Wp, bp)[0]


def _tc_init(t, embrows, dcol, W0):
    return pl.pallas_call(
        _tc_init_body,
        grid=(N // _BLK,),
        in_specs=[_row_spec(H), _row_spec(H), _row_spec(1),
                  _full_spec(H, H)],
        out_specs=[_row_spec(H), _row_spec(H)],
        out_shape=[jax.ShapeDtypeStruct((N, H), jnp.float32)] * 2,
    )(t, embrows, dcol, W0)


def _tc_layer(h, hwp, p0, p1, dcol, R, rb, scale, shift, Wn):
    return pl.pallas_call(
        _tc_layer_body,
        grid=(N // _BLK,),
        in_specs=[_row_spec(H)] * 4 + [_row_spec(1)] +
                 [_full_spec(H, H), _full_spec(1, H), _full_spec(1, H),
                  _full_spec(1, H), _full_spec(H, H)],
        out_specs=[_row_spec(H), _row_spec(H)],
        out_shape=[jax.ShapeDtypeStruct((N, H), jnp.float32)] * 2,
    )(h, hwp, p0, p1, dcol, R, rb, scale, shift, Wn)


def _tc_pool(h, hwp, p0, p1, dcol, R, rb, scale, shift,
             batch2d, Wout, bout, ln_g, ln_b):
    return pl.pallas_call(
        _tc_pool_body,
        grid=(N // _BLK,),
        in_specs=[_row_spec(H)] * 4 + [_row_spec(1)] +
                 [_full_spec(H, H), _full_spec(1, H), _full_spec(1, H),
                  _full_spec(1, H)] +
                 [_row_spec(1),
                  _full_spec(2 * H, OUT), _full_spec(1, OUT),
                  _full_spec(1, OUT), _full_spec(1, OUT)],
        out_specs=[_full_spec(G, OUT)],
        out_shape=[jax.ShapeDtypeStruct((G, OUT), jnp.float32)],
        scratch_shapes=[pltpu.VMEM((G, H), jnp.float32),
                        pltpu.VMEM((G, H), jnp.float32),
                        pltpu.VMEM((G, H), jnp.float32)],
    )(h, hwp, p0, p1, dcol, R, rb, scale, shift,
      batch2d, Wout, bout, ln_g, ln_b)[0]


def kernel(x, node_ids, edge_index, batch, emb, Wp, bp,
           W0, b0, G0, B0, R0, rb0,
           W1, b1, G1, B1, R1, rb1,
           W2, b2, G2, B2, R2, rb2,
           Wout, bout, ln_g, ln_b):
    f32 = jnp.float32
    src = edge_index[0].astype(jnp.int32)
    dst = edge_index[1].astype(jnp.int32)
    pad_i = jnp.arange(E_PAD - E, dtype=jnp.int32)
    src_p = jnp.concatenate([src, pad_i % N])
    dst_p = jnp.concatenate([dst, DUMMY + pad_i % (ACC_ROWS - N)])
    ids_p = jnp.concatenate(
        [node_ids.astype(jnp.int32), jnp.zeros((NID_PAD - N,), jnp.int32)])

    ohH = jnp.concatenate(
        [jnp.ones((CH, 1), f32), jnp.zeros((CH, H - 1), f32)], axis=1)
    zH = jnp.zeros((CH, H), f32)

    cbn = 1.0 / math.sqrt(1.0 + EPS)
    bp2 = bp.reshape(1, H).astype(f32)
    scales = [(cbn * g).reshape(1, H) for g in (G0, G1, G2)]
    shifts = [(b * cbn * g + bb).reshape(1, H)
              for (b, g, bb) in ((b0, G0, B0), (b1, G1, B1), (b2, G2, B2))]

    t = _tc_proj(x, Wp, bp2)
    embrows, degp = _get_sc_pre()(ids_p, dst_p, ohH, zH, emb)
    embrows = embrows[:N]
    dcol = degp[0, :N, 0:1] + degp[1, :N, 0:1]

    h, hwp = _tc_init(t, embrows, dcol, W0)

    parts = _get_sc_spmm()(src_p, dst_p, zH, hwp)
    h, hwp = _tc_layer(h, hwp, parts[0, :N], parts[1, :N], dcol,
                       R0, rb0.reshape(1, H), scales[0], shifts[0], W1)

    parts = _get_sc_spmm()(src_p, dst_p, zH, hwp)
    h, hwp = _tc_layer(h, hwp, parts[0, :N], parts[1, :N], dcol,
                       R1, rb1.reshape(1, H), scales[1], shifts[1], W2)

    parts = _get_sc_spmm()(src_p, dst_p, zH, hwp)
    batch2d = batch.astype(jnp.int32).reshape(N, 1)
    return _tc_pool(h, hwp, parts[0, :N], parts[1, :N], dcol,
                    R2, rb2.reshape(1, H), scales[2], shifts[2],
                    batch2d, Wout, bout.reshape(1, OUT),
                    ln_g.reshape(1, OUT), ln_b.reshape(1, OUT))
